# Initial kernel scaffold; baseline (speedup 1.0000x reference)
import jax
import jax.numpy as jnp
from jax.experimental import pallas as pl


def kernel(x, edge_index, edge_attr, batch, We1, be1, Wn1, bn1, We2, be2, Wn2, bn2, Wg1, bg1, Wg2, bg2):
    raise NotImplementedError("write your pallas kernel here")


# speedup vs baseline: 3.8583x; 3.8583x over previous
import functools

import jax
import jax.numpy as jnp
from jax import lax
from jax.experimental import pallas as pl
from jax.experimental.pallas import tpu as pltpu
from jax.experimental.pallas import tpu_sc as plsc

N_NODES = 10000
N_EDGES = 320000
D_FEAT = 128
D_EDGE = 16
MSG = 32
HID = 128
OUT = 16
N_GRAPHS = 64

NC = 2
NS = 16
NW = NC * NS
EPW = N_EDGES // NW
K = 80
NPAD = 10240
NPS = NPAD // NS


_EBLK = 8000


def _edge_pre_body(ea_ref, w1_ref, b1_ref, w2_ref, b2_ref, e1_ref, e2_ref):
    ea = ea_ref[...]
    e1_ref[...] = jnp.dot(ea, w1_ref[...], preferred_element_type=jnp.float32) + b1_ref[...]
    e2_ref[...] = jnp.dot(ea, w2_ref[...], preferred_element_type=jnp.float32) + b2_ref[...]


def _edge_pre(ea, w1t, b1, w2t, b2):
    nblk = N_EDGES // _EBLK
    return pl.pallas_call(
        _edge_pre_body,
        grid=(nblk,),
        in_specs=[
            pl.BlockSpec((_EBLK, D_EDGE), lambda i: (i, 0)),
            pl.BlockSpec((D_EDGE, MSG), lambda i: (0, 0)),
            pl.BlockSpec((1, MSG), lambda i: (0, 0)),
            pl.BlockSpec((D_EDGE, MSG), lambda i: (0, 0)),
            pl.BlockSpec((1, MSG), lambda i: (0, 0)),
        ],
        out_specs=[
            pl.BlockSpec((_EBLK, MSG), lambda i: (i, 0)),
            pl.BlockSpec((_EBLK, MSG), lambda i: (i, 0)),
        ],
        out_shape=[
            jax.ShapeDtypeStruct((N_EDGES, MSG), jnp.float32),
            jax.ShapeDtypeStruct((N_EDGES, MSG), jnp.float32),
        ],
    )(ea, w1t, b1, w2t, b2)


_NBLK = 2000


def _proj_body(x_ref, wa_ref, wb_ref, p0_ref, p1_ref):
    xv = x_ref[...]
    p0_ref[...] = jnp.dot(xv, wa_ref[...], preferred_element_type=jnp.float32)
    p1_ref[...] = jnp.dot(xv, wb_ref[...], preferred_element_type=jnp.float32)


def _proj(x, wat, wbt):
    nblk = N_NODES // _NBLK
    return pl.pallas_call(
        _proj_body,
        grid=(nblk,),
        in_specs=[
            pl.BlockSpec((_NBLK, D_FEAT), lambda i: (i, 0)),
            pl.BlockSpec((D_FEAT, MSG), lambda i: (0, 0)),
            pl.BlockSpec((D_FEAT, MSG), lambda i: (0, 0)),
        ],
        out_specs=[
            pl.BlockSpec((_NBLK, MSG), lambda i: (i, 0)),
            pl.BlockSpec((_NBLK, MSG), lambda i: (i, 0)),
        ],
        out_shape=[
            jax.ShapeDtypeStruct((N_NODES, MSG), jnp.float32),
            jax.ShapeDtypeStruct((N_NODES, MSG), jnp.float32),
        ],
    )(x, wat, wbt)


def _sc_conv_body(p0_hbm, p1_hbm, e_hbm, n0_hbm, n1_hbm, z_hbm, out_hbm,
                  agg_sh, idx0_v, idx1_v, g0_v, g1_v, ev_v, sem):
    c = lax.axis_index("c")
    s = lax.axis_index("s")
    wid = c * NS + s

    pltpu.sync_copy(z_hbm.at[pl.ds(s * NPS, NPS)], agg_sh.at[pl.ds(s * NPS, NPS)])
    plsc.subcore_barrier()

    base0 = wid * EPW

    def step(t, carry):
        base = base0 + t * K
        pltpu.sync_copy(n0_hbm.at[pl.ds(base, K)], idx0_v)
        pltpu.sync_copy(n1_hbm.at[pl.ds(base, K)], idx1_v)
        pltpu.sync_copy(e_hbm.at[pl.ds(base, K)], ev_v)
        pltpu.async_copy(p0_hbm.at[idx0_v], g0_v, sem).wait()
        pltpu.async_copy(p1_hbm.at[idx1_v], g1_v, sem).wait()

        def row(r, carry2):
            for h in range(MSG // 16):
                sl = pl.ds(h * 16, 16)
                v = g0_v[r, sl] + g1_v[r, sl] + ev_v[r, sl]
                ev_v[r, sl] = jnp.maximum(v, 0.0)
            return carry2

        lax.fori_loop(0, K, row, 0)
        pltpu.sync_copy(ev_v, agg_sh.at[idx0_v], add=True)
        return carry

    lax.fori_loop(0, EPW // K, step, 0)
    plsc.subcore_barrier()
    pltpu.sync_copy(agg_sh.at[pl.ds(s * NPS, NPS)],
                    out_hbm.at[c, pl.ds(s * NPS, NPS)])


def _sc_conv(p0, p1, e, n0, n1, zeros):
    mesh = plsc.VectorSubcoreMesh(core_axis_name="c", subcore_axis_name="s")
    f = pl.kernel(
        _sc_conv_body,
        out_type=jax.ShapeDtypeStruct((NC, NPAD, MSG), jnp.float32),
        mesh=mesh,
        scratch_types=[
            pltpu.VMEM_SHARED((NPAD, MSG), jnp.float32),
            pltpu.VMEM((K,), jnp.int32),
            pltpu.VMEM((K,), jnp.int32),
            pltpu.VMEM((K, MSG), jnp.float32),
            pltpu.VMEM((K, MSG), jnp.float32),
            pltpu.VMEM((K, MSG), jnp.float32),
            pltpu.SemaphoreType.DMA,
        ],
        compiler_params=pltpu.CompilerParams(use_tc_tiling_on_sc=False),
    )
    return f(p0, p1, e, n0, n1, zeros)


_UBLK = 2000


def _node_up_body(x_ref, agg_ref, wna_ref, wnb_ref, bn_ref, wa2_ref, wb2_ref,
                  h_ref, p0_ref, p1_ref):
    aggs = agg_ref[0] + agg_ref[1]
    h = jnp.dot(x_ref[...], wna_ref[...], preferred_element_type=jnp.float32)
    h += jnp.dot(aggs, wnb_ref[...], preferred_element_type=jnp.float32)
    h = jnp.maximum(h + bn_ref[...], 0.0)
    h_ref[...] = h
    p0_ref[...] = jnp.dot(h, wa2_ref[...], preferred_element_type=jnp.float32)
    p1_ref[...] = jnp.dot(h, wb2_ref[...], preferred_element_type=jnp.float32)


def _node_update(x, agg, wnat, wnbt, bn, wa2t, wb2t):
    nblk = N_NODES // _UBLK
    return pl.pallas_call(
        _node_up_body,
        grid=(nblk,),
        in_specs=[
            pl.BlockSpec((_UBLK, D_FEAT), lambda i: (i, 0)),
            pl.BlockSpec((NC, _UBLK, MSG), lambda i: (0, i, 0)),
            pl.BlockSpec((D_FEAT, D_FEAT), lambda i: (0, 0)),
            pl.BlockSpec((MSG, D_FEAT), lambda i: (0, 0)),
            pl.BlockSpec((1, D_FEAT), lambda i: (0, 0)),
            pl.BlockSpec((D_FEAT, MSG), lambda i: (0, 0)),
            pl.BlockSpec((D_FEAT, MSG), lambda i: (0, 0)),
        ],
        out_specs=[
            pl.BlockSpec((_UBLK, D_FEAT), lambda i: (i, 0)),
            pl.BlockSpec((_UBLK, MSG), lambda i: (i, 0)),
            pl.BlockSpec((_UBLK, MSG), lambda i: (i, 0)),
        ],
        out_shape=[
            jax.ShapeDtypeStruct((N_NODES, D_FEAT), jnp.float32),
            jax.ShapeDtypeStruct((N_NODES, MSG), jnp.float32),
            jax.ShapeDtypeStruct((N_NODES, MSG), jnp.float32),
        ],
    )(x, agg, wnat, wnbt, bn, wa2t, wb2t)


_PBLK = 2000


def _pool_body(x_ref, agg_ref, batch_ref, wna_ref, wnb_ref, bn_ref,
               wg1_ref, bg1_ref, wg2_ref, bg2_ref, z_ref, sum_acc, cnt_acc):
    i = pl.program_id(0)
    aggs = agg_ref[0] + agg_ref[1]
    h = jnp.dot(x_ref[...], wna_ref[...], preferred_element_type=jnp.float32)
    h += jnp.dot(aggs, wnb_ref[...], preferred_element_type=jnp.float32)
    h = jnp.maximum(h + bn_ref[...], 0.0)

    b = batch_ref[0, 0, :]
    gids = lax.broadcasted_iota(jnp.int32, (N_GRAPHS, _PBLK), 0)
    onehot = (gids == b[None, :]).astype(jnp.float32)

    @pl.when(i == 0)
    def _init():
        sum_acc[...] = jnp.zeros_like(sum_acc)
        cnt_acc[...] = jnp.zeros_like(cnt_acc)

    sum_acc[...] += jnp.dot(onehot, h, preferred_element_type=jnp.float32)
    cnt_acc[...] += jnp.sum(onehot, axis=1, keepdims=True)

    @pl.when(i == pl.num_programs(0) - 1)
    def _final():
        means = sum_acc[...] / jnp.clip(cnt_acc[...], 1.0, None)
        g = jnp.dot(means, wg1_ref[...], preferred_element_type=jnp.float32)
        g = jnp.maximum(g + bg1_ref[...], 0.0)
        z = jnp.dot(g, wg2_ref[...], preferred_element_type=jnp.float32)
        z_ref[...] = z + bg2_ref[...]


def _pool_mlp(x, agg, batch3d, wnat, wnbt, bn, wg1t, bg1, wg2t, bg2):
    nblk = N_NODES // _PBLK
    return pl.pallas_call(
        _pool_body,
        grid=(nblk,),
        in_specs=[
            pl.BlockSpec((_PBLK, D_FEAT), lambda i: (i, 0)),
            pl.BlockSpec((NC, _PBLK, MSG), lambda i: (0, i, 0)),
            pl.BlockSpec((1, 1, _PBLK), lambda i: (i, 0, 0)),
            pl.BlockSpec((D_FEAT, D_FEAT), lambda i: (0, 0)),
            pl.BlockSpec((MSG, D_FEAT), lambda i: (0, 0)),
            pl.BlockSpec((1, D_FEAT), lambda i: (0, 0)),
            pl.BlockSpec((D_FEAT, HID), lambda i: (0, 0)),
            pl.BlockSpec((1, HID), lambda i: (0, 0)),
            pl.BlockSpec((HID, OUT), lambda i: (0, 0)),
            pl.BlockSpec((1, OUT), lambda i: (0, 0)),
        ],
        out_specs=pl.BlockSpec((N_GRAPHS, OUT), lambda i: (0, 0)),
        out_shape=jax.ShapeDtypeStruct((N_GRAPHS, OUT), jnp.float32),
        scratch_shapes=[
            pltpu.VMEM((N_GRAPHS, D_FEAT), jnp.float32),
            pltpu.VMEM((N_GRAPHS, 1), jnp.float32),
        ],
    )(x, agg, batch3d, wnat, wnbt, bn, wg1t, bg1, wg2t, bg2)


def kernel(x, edge_index, edge_attr, batch,
           We1, be1, Wn1, bn1, We2, be2, Wn2, bn2, Wg1, bg1, Wg2, bg2):
    n0 = edge_index[0]
    n1 = edge_index[1]

    wa1t = We1[:, :D_FEAT].T
    wb1t = We1[:, D_FEAT:2 * D_FEAT].T
    wc1t = We1[:, 2 * D_FEAT:].T
    wa2t = We2[:, :D_FEAT].T
    wb2t = We2[:, D_FEAT:2 * D_FEAT].T
    wc2t = We2[:, 2 * D_FEAT:].T
    wna1t = Wn1[:, :D_FEAT].T
    wnb1t = Wn1[:, D_FEAT:].T
    wna2t = Wn2[:, :D_FEAT].T
    wnb2t = Wn2[:, D_FEAT:].T
    wg1t = Wg1.T
    wg2t = Wg2.T

    be1r = be1.reshape(1, MSG)
    be2r = be2.reshape(1, MSG)
    bn1r = bn1.reshape(1, D_FEAT)
    bn2r = bn2.reshape(1, D_FEAT)
    bg1r = bg1.reshape(1, HID)
    bg2r = bg2.reshape(1, OUT)

    zeros = jnp.zeros((NPAD, MSG), jnp.float32)
    batch3d = batch.reshape(N_NODES // _PBLK, 1, _PBLK)

    e1, e2 = _edge_pre(edge_attr, wc1t, be1r, wc2t, be2r)
    p0, p1 = _proj(x, wa1t, wb1t)
    agg1 = _sc_conv(p0, p1, e1, n0, n1, zeros)
    h1, q0, q1 = _node_update(x, agg1, wna1t, wnb1t, bn1r, wa2t, wb2t)

    agg2 = _sc_conv(q0, q1, e2, n0, n1, zeros)

    z = _pool_mlp(h1, agg2, batch3d, wna2t, wnb2t, bn2r, wg1t, bg1r, wg2t, bg2r)
    return z


# speedup vs baseline: 6.5759x; 1.7043x over previous
import functools

import jax
import jax.numpy as jnp
from jax import lax
from jax.experimental import pallas as pl
from jax.experimental.pallas import tpu as pltpu
from jax.experimental.pallas import tpu_sc as plsc

N_NODES = 10000
N_EDGES = 320000
D_FEAT = 128
D_EDGE = 16
MSG = 32
HID = 128
OUT = 16
N_GRAPHS = 64

NC = 2
NS = 16
NW = NC * NS
EPW = N_EDGES // NW
K = 80
NPAD = 10240
NPS = NPAD // NS


_EBLK = 8000


def _edge_pre_body(ea_ref, w1_ref, b1_ref, w2_ref, b2_ref, e1_ref, e2_ref):
    ea = ea_ref[...]
    e1_ref[...] = jnp.dot(ea, w1_ref[...], preferred_element_type=jnp.float32) + b1_ref[...]
    e2_ref[...] = jnp.dot(ea, w2_ref[...], preferred_element_type=jnp.float32) + b2_ref[...]


def _edge_pre(ea, w1t, b1, w2t, b2):
    nblk = N_EDGES // _EBLK
    return pl.pallas_call(
        _edge_pre_body,
        grid=(nblk,),
        in_specs=[
            pl.BlockSpec((_EBLK, D_EDGE), lambda i: (i, 0)),
            pl.BlockSpec((D_EDGE, MSG), lambda i: (0, 0)),
            pl.BlockSpec((1, MSG), lambda i: (0, 0)),
            pl.BlockSpec((D_EDGE, MSG), lambda i: (0, 0)),
            pl.BlockSpec((1, MSG), lambda i: (0, 0)),
        ],
        out_specs=[
            pl.BlockSpec((_EBLK, MSG), lambda i: (i, 0)),
            pl.BlockSpec((_EBLK, MSG), lambda i: (i, 0)),
        ],
        out_shape=[
            jax.ShapeDtypeStruct((N_EDGES, MSG), jnp.float32),
            jax.ShapeDtypeStruct((N_EDGES, MSG), jnp.float32),
        ],
    )(ea, w1t, b1, w2t, b2)


_NBLK = 2000


def _proj_body(x_ref, wa_ref, wb_ref, p0_ref, p1_ref):
    xv = x_ref[...]
    p0_ref[...] = jnp.dot(xv, wa_ref[...], preferred_element_type=jnp.float32)
    p1_ref[...] = jnp.dot(xv, wb_ref[...], preferred_element_type=jnp.float32)


def _proj(x, wat, wbt):
    nblk = N_NODES // _NBLK
    return pl.pallas_call(
        _proj_body,
        grid=(nblk,),
        in_specs=[
            pl.BlockSpec((_NBLK, D_FEAT), lambda i: (i, 0)),
            pl.BlockSpec((D_FEAT, MSG), lambda i: (0, 0)),
            pl.BlockSpec((D_FEAT, MSG), lambda i: (0, 0)),
        ],
        out_specs=[
            pl.BlockSpec((_NBLK, MSG), lambda i: (i, 0)),
            pl.BlockSpec((_NBLK, MSG), lambda i: (i, 0)),
        ],
        out_shape=[
            jax.ShapeDtypeStruct((N_NODES, MSG), jnp.float32),
            jax.ShapeDtypeStruct((N_NODES, MSG), jnp.float32),
        ],
    )(x, wat, wbt)


SUP = 400
NT = SUP // K
T_STEPS = EPW // SUP
ROWS2 = N_EDGES // K


def _sc_conv_body(p0_hbm, p1_hbm, e_hbm, n0_hbm, n1_hbm, z_hbm, out_hbm,
                  agg_sh, idx0_v, idx1_v, g0_v, g1_v, ev_v,
                  sem_i, sem_g, sem_s):
    c = lax.axis_index("c")
    s = lax.axis_index("s")
    wid = c * NS + s

    pltpu.sync_copy(z_hbm.at[pl.ds(s * NPS, NPS)], agg_sh.at[pl.ds(s * NPS, NPS)])

    irow0 = wid * (EPW // K)
    base0 = wid * EPW

    def issue_idx(t, slot):
        r = irow0 + t * NT
        pltpu.async_copy(n0_hbm.at[pl.ds(r, NT)], idx0_v.at[slot], sem_i.at[slot])
        pltpu.async_copy(n1_hbm.at[pl.ds(r, NT)], idx1_v.at[slot], sem_i.at[slot])

    def drain_idx(t, slot):
        r = irow0 + t * NT
        pltpu.make_async_copy(n0_hbm.at[pl.ds(r, NT)], idx0_v.at[slot], sem_i.at[slot]).wait()
        pltpu.make_async_copy(n1_hbm.at[pl.ds(r, NT)], idx1_v.at[slot], sem_i.at[slot]).wait()

    def issue_fetch(t, b, slot):
        base = base0 + t * SUP
        pltpu.async_copy(e_hbm.at[pl.ds(base, SUP)], ev_v.at[b], sem_g.at[b])
        for j in range(NT):
            sl = pl.ds(j * K, K)
            pltpu.async_copy(p0_hbm.at[idx0_v.at[slot, j]], g0_v.at[b, sl], sem_g.at[b])
            pltpu.async_copy(p1_hbm.at[idx1_v.at[slot, j]], g1_v.at[b, sl], sem_g.at[b])

    def drain_fetch(t, b):
        base = base0 + t * SUP
        pltpu.make_async_copy(e_hbm.at[pl.ds(base, SUP)], ev_v.at[b], sem_g.at[b]).wait()
        pltpu.make_async_copy(e_hbm.at[pl.ds(base, SUP)], g0_v.at[b], sem_g.at[b]).wait()
        pltpu.make_async_copy(e_hbm.at[pl.ds(base, SUP)], g1_v.at[b], sem_g.at[b]).wait()

    def issue_scatter(b, slot):
        for j in range(NT):
            sl = pl.ds(j * K, K)
            pltpu.make_async_copy(ev_v.at[b, sl], agg_sh.at[idx0_v.at[slot, j]],
                                  sem_s.at[b]).start(add=True)

    def drain_scatter(b, slot):
        for j in range(NT):
            sl = pl.ds(j * K, K)
            pltpu.make_async_copy(ev_v.at[b, sl], agg_sh.at[idx0_v.at[slot, j]],
                                  sem_s.at[b]).wait()

    issue_idx(0, 0)
    issue_idx(1, 1)
    drain_idx(0, 0)
    issue_fetch(0, 0, 0)

    def step(t, carry):
        b = t % 2
        slot = t % 3

        @pl.when(t >= 1)
        def _():
            drain_scatter(1 - b, (t - 1) % 3)

        @pl.when(t + 2 < T_STEPS)
        def _():
            issue_idx(t + 2, (t + 2) % 3)

        @pl.when(t + 1 < T_STEPS)
        def _():
            drain_idx(t + 1, (t + 1) % 3)
            issue_fetch(t + 1, 1 - b, (t + 1) % 3)

        drain_fetch(t, b)

        def row4(u, carry2):
            for k in range(4):
                r = u * 4 + k
                for h in range(MSG // 16):
                    sl = pl.ds(h * 16, 16)
                    v = g0_v[b, r, sl] + g1_v[b, r, sl] + ev_v[b, r, sl]
                    ev_v[b, r, sl] = jnp.maximum(v, 0.0)
            return carry2

        lax.fori_loop(0, SUP // 4, row4, 0)
        issue_scatter(b, slot)
        return carry

    lax.fori_loop(0, T_STEPS, step, 0)
    drain_scatter((T_STEPS - 1) % 2, (T_STEPS - 1) % 3)
    plsc.subcore_barrier()
    pltpu.sync_copy(agg_sh.at[pl.ds(s * NPS, NPS)],
                    out_hbm.at[c, pl.ds(s * NPS, NPS)])


def _sc_conv(p0, p1, e, n0, n1, zeros):
    mesh = plsc.VectorSubcoreMesh(core_axis_name="c", subcore_axis_name="s")
    f = pl.kernel(
        _sc_conv_body,
        out_type=jax.ShapeDtypeStruct((NC, NPAD, MSG), jnp.float32),
        mesh=mesh,
        scratch_types=[
            pltpu.VMEM_SHARED((NPAD, MSG), jnp.float32),
            pltpu.VMEM((3, NT, K), jnp.int32),
            pltpu.VMEM((3, NT, K), jnp.int32),
            pltpu.VMEM((2, SUP, MSG), jnp.float32),
            pltpu.VMEM((2, SUP, MSG), jnp.float32),
            pltpu.VMEM((2, SUP, MSG), jnp.float32),
            pltpu.SemaphoreType.DMA((3,)),
            pltpu.SemaphoreType.DMA((2,)),
            pltpu.SemaphoreType.DMA((2,)),
        ],
        compiler_params=pltpu.CompilerParams(use_tc_tiling_on_sc=False),
    )
    return f(p0, p1, e, n0, n1, zeros)


_UBLK = 2000


def _node_up_body(x_ref, agg_ref, wna_ref, wnb_ref, bn_ref, wa2_ref, wb2_ref,
                  h_ref, p0_ref, p1_ref):
    aggs = agg_ref[0] + agg_ref[1]
    h = jnp.dot(x_ref[...], wna_ref[...], preferred_element_type=jnp.float32)
    h += jnp.dot(aggs, wnb_ref[...], preferred_element_type=jnp.float32)
    h = jnp.maximum(h + bn_ref[...], 0.0)
    h_ref[...] = h
    p0_ref[...] = jnp.dot(h, wa2_ref[...], preferred_element_type=jnp.float32)
    p1_ref[...] = jnp.dot(h, wb2_ref[...], preferred_element_type=jnp.float32)


def _node_update(x, agg, wnat, wnbt, bn, wa2t, wb2t):
    nblk = N_NODES // _UBLK
    return pl.pallas_call(
        _node_up_body,
        grid=(nblk,),
        in_specs=[
            pl.BlockSpec((_UBLK, D_FEAT), lambda i: (i, 0)),
            pl.BlockSpec((NC, _UBLK, MSG), lambda i: (0, i, 0)),
            pl.BlockSpec((D_FEAT, D_FEAT), lambda i: (0, 0)),
            pl.BlockSpec((MSG, D_FEAT), lambda i: (0, 0)),
            pl.BlockSpec((1, D_FEAT), lambda i: (0, 0)),
            pl.BlockSpec((D_FEAT, MSG), lambda i: (0, 0)),
            pl.BlockSpec((D_FEAT, MSG), lambda i: (0, 0)),
        ],
        out_specs=[
            pl.BlockSpec((_UBLK, D_FEAT), lambda i: (i, 0)),
            pl.BlockSpec((_UBLK, MSG), lambda i: (i, 0)),
            pl.BlockSpec((_UBLK, MSG), lambda i: (i, 0)),
        ],
        out_shape=[
            jax.ShapeDtypeStruct((N_NODES, D_FEAT), jnp.float32),
            jax.ShapeDtypeStruct((N_NODES, MSG), jnp.float32),
            jax.ShapeDtypeStruct((N_NODES, MSG), jnp.float32),
        ],
    )(x, agg, wnat, wnbt, bn, wa2t, wb2t)


_PBLK = 2000


def _pool_body(x_ref, agg_ref, batch_ref, wna_ref, wnb_ref, bn_ref,
               wg1_ref, bg1_ref, wg2_ref, bg2_ref, z_ref, sum_acc, cnt_acc):
    i = pl.program_id(0)
    aggs = agg_ref[0] + agg_ref[1]
    h = jnp.dot(x_ref[...], wna_ref[...], preferred_element_type=jnp.float32)
    h += jnp.dot(aggs, wnb_ref[...], preferred_element_type=jnp.float32)
    h = jnp.maximum(h + bn_ref[...], 0.0)

    b = batch_ref[0, 0, :]
    gids = lax.broadcasted_iota(jnp.int32, (N_GRAPHS, _PBLK), 0)
    onehot = (gids == b[None, :]).astype(jnp.float32)

    @pl.when(i == 0)
    def _init():
        sum_acc[...] = jnp.zeros_like(sum_acc)
        cnt_acc[...] = jnp.zeros_like(cnt_acc)

    sum_acc[...] += jnp.dot(onehot, h, preferred_element_type=jnp.float32)
    cnt_acc[...] += jnp.sum(onehot, axis=1, keepdims=True)

    @pl.when(i == pl.num_programs(0) - 1)
    def _final():
        means = sum_acc[...] / jnp.clip(cnt_acc[...], 1.0, None)
        g = jnp.dot(means, wg1_ref[...], preferred_element_type=jnp.float32)
        g = jnp.maximum(g + bg1_ref[...], 0.0)
        z = jnp.dot(g, wg2_ref[...], preferred_element_type=jnp.float32)
        z_ref[...] = z + bg2_ref[...]


def _pool_mlp(x, agg, batch3d, wnat, wnbt, bn, wg1t, bg1, wg2t, bg2):
    nblk = N_NODES // _PBLK
    return pl.pallas_call(
        _pool_body,
        grid=(nblk,),
        in_specs=[
            pl.BlockSpec((_PBLK, D_FEAT), lambda i: (i, 0)),
            pl.BlockSpec((NC, _PBLK, MSG), lambda i: (0, i, 0)),
            pl.BlockSpec((1, 1, _PBLK), lambda i: (i, 0, 0)),
            pl.BlockSpec((D_FEAT, D_FEAT), lambda i: (0, 0)),
            pl.BlockSpec((MSG, D_FEAT), lambda i: (0, 0)),
            pl.BlockSpec((1, D_FEAT), lambda i: (0, 0)),
            pl.BlockSpec((D_FEAT, HID), lambda i: (0, 0)),
            pl.BlockSpec((1, HID), lambda i: (0, 0)),
            pl.BlockSpec((HID, OUT), lambda i: (0, 0)),
            pl.BlockSpec((1, OUT), lambda i: (0, 0)),
        ],
        out_specs=pl.BlockSpec((N_GRAPHS, OUT), lambda i: (0, 0)),
        out_shape=jax.ShapeDtypeStruct((N_GRAPHS, OUT), jnp.float32),
        scratch_shapes=[
            pltpu.VMEM((N_GRAPHS, D_FEAT), jnp.float32),
            pltpu.VMEM((N_GRAPHS, 1), jnp.float32),
        ],
    )(x, agg, batch3d, wnat, wnbt, bn, wg1t, bg1, wg2t, bg2)


def kernel(x, edge_index, edge_attr, batch,
           We1, be1, Wn1, bn1, We2, be2, Wn2, bn2, Wg1, bg1, Wg2, bg2):
    n0 = edge_index[0].reshape(ROWS2, K)
    n1 = edge_index[1].reshape(ROWS2, K)

    wa1t = We1[:, :D_FEAT].T
    wb1t = We1[:, D_FEAT:2 * D_FEAT].T
    wc1t = We1[:, 2 * D_FEAT:].T
    wa2t = We2[:, :D_FEAT].T
    wb2t = We2[:, D_FEAT:2 * D_FEAT].T
    wc2t = We2[:, 2 * D_FEAT:].T
    wna1t = Wn1[:, :D_FEAT].T
    wnb1t = Wn1[:, D_FEAT:].T
    wna2t = Wn2[:, :D_FEAT].T
    wnb2t = Wn2[:, D_FEAT:].T
    wg1t = Wg1.T
    wg2t = Wg2.T

    be1r = be1.reshape(1, MSG)
    be2r = be2.reshape(1, MSG)
    bn1r = bn1.reshape(1, D_FEAT)
    bn2r = bn2.reshape(1, D_FEAT)
    bg1r = bg1.reshape(1, HID)
    bg2r = bg2.reshape(1, OUT)

    zeros = jnp.zeros((NPAD, MSG), jnp.float32)
    batch3d = batch.reshape(N_NODES // _PBLK, 1, _PBLK)

    e1, e2 = _edge_pre(edge_attr, wc1t, be1r, wc2t, be2r)
    p0, p1 = _proj(x, wa1t, wb1t)
    agg1 = _sc_conv(p0, p1, e1, n0, n1, zeros)
    h1, q0, q1 = _node_update(x, agg1, wna1t, wnb1t, bn1r, wa2t, wb2t)

    agg2 = _sc_conv(q0, q1, e2, n0, n1, zeros)

    z = _pool_mlp(h1, agg2, batch3d, wna2t, wnb2t, bn2r, wg1t, bg1r, wg2t, bg2r)
    return z


# speedup vs baseline: 8.3911x; 1.2760x over previous
import functools

import jax
import jax.numpy as jnp
from jax import lax
from jax.experimental import pallas as pl
from jax.experimental.pallas import tpu as pltpu
from jax.experimental.pallas import tpu_sc as plsc

N_NODES = 10000
N_EDGES = 320000
D_FEAT = 128
D_EDGE = 16
MSG = 32
HID = 128
OUT = 16
N_GRAPHS = 64

NC = 2
NS = 16
NW = NC * NS
EPW = N_EDGES // NW
K = 80
NPAD = 10240
NPS = NPAD // NS


EROWS = N_EDGES // 8
_EBLK = 2000


def _edge_pre_body(ea_ref, w1_ref, b1_ref, w2_ref, b2_ref, e1_ref, e2_ref):
    ea = ea_ref[...]
    e1_ref[...] = jnp.dot(ea, w1_ref[...], preferred_element_type=jnp.float32) + b1_ref[...]
    e2_ref[...] = jnp.dot(ea, w2_ref[...], preferred_element_type=jnp.float32) + b2_ref[...]


def _edge_pre(ea8, w1bd, b1, w2bd, b2):
    nblk = EROWS // _EBLK
    return pl.pallas_call(
        _edge_pre_body,
        grid=(nblk,),
        in_specs=[
            pl.BlockSpec((_EBLK, 128), lambda i: (i, 0)),
            pl.BlockSpec((128, 8 * MSG), lambda i: (0, 0)),
            pl.BlockSpec((1, 8 * MSG), lambda i: (0, 0)),
            pl.BlockSpec((128, 8 * MSG), lambda i: (0, 0)),
            pl.BlockSpec((1, 8 * MSG), lambda i: (0, 0)),
        ],
        out_specs=[
            pl.BlockSpec((_EBLK, 8 * MSG), lambda i: (i, 0)),
            pl.BlockSpec((_EBLK, 8 * MSG), lambda i: (i, 0)),
        ],
        out_shape=[
            jax.ShapeDtypeStruct((EROWS, 8 * MSG), jnp.float32),
            jax.ShapeDtypeStruct((EROWS, 8 * MSG), jnp.float32),
        ],
    )(ea8, w1bd, b1, w2bd, b2)


_NBLK = 2000


def _proj_body(x_ref, wa_ref, wb_ref, p0_ref, p1_ref):
    xv = x_ref[...]
    p0_ref[...] = jnp.dot(xv, wa_ref[...], preferred_element_type=jnp.float32)
    p1_ref[...] = jnp.dot(xv, wb_ref[...], preferred_element_type=jnp.float32)


def _proj(x, wat, wbt):
    nblk = N_NODES // _NBLK
    return pl.pallas_call(
        _proj_body,
        grid=(nblk,),
        in_specs=[
            pl.BlockSpec((_NBLK, D_FEAT), lambda i: (i, 0)),
            pl.BlockSpec((D_FEAT, MSG), lambda i: (0, 0)),
            pl.BlockSpec((D_FEAT, MSG), lambda i: (0, 0)),
        ],
        out_specs=[
            pl.BlockSpec((_NBLK, MSG), lambda i: (i, 0)),
            pl.BlockSpec((_NBLK, MSG), lambda i: (i, 0)),
        ],
        out_shape=[
            jax.ShapeDtypeStruct((N_NODES, MSG), jnp.float32),
            jax.ShapeDtypeStruct((N_NODES, MSG), jnp.float32),
        ],
    )(x, wat, wbt)


SUP = 400
NT = SUP // K
T_STEPS = EPW // SUP
ROWS2 = N_EDGES // K


def _sc_conv_body(p0_hbm, p1_hbm, e_hbm, n0_hbm, n1_hbm, z_hbm, out_hbm,
                  agg_sh, idx0_v, idx1_v, g0_v, g1_v, ev_v, m_v,
                  sem_i, sem_g, sem_s):
    c = lax.axis_index("c")
    s = lax.axis_index("s")
    wid = c * NS + s

    pltpu.sync_copy(z_hbm.at[pl.ds(s * NPS, NPS)], agg_sh.at[pl.ds(s * NPS, NPS)])
    plsc.subcore_barrier()

    irow0 = wid * (EPW // K)
    base0 = wid * EPW

    def issue_idx(t, slot):
        r = irow0 + t * NT
        pltpu.async_copy(n0_hbm.at[pl.ds(r, NT)], idx0_v.at[slot], sem_i.at[slot])
        pltpu.async_copy(n1_hbm.at[pl.ds(r, NT)], idx1_v.at[slot], sem_i.at[slot])

    def drain_idx(t, slot):
        r = irow0 + t * NT
        pltpu.make_async_copy(n0_hbm.at[pl.ds(r, NT)], idx0_v.at[slot], sem_i.at[slot]).wait()
        pltpu.make_async_copy(n1_hbm.at[pl.ds(r, NT)], idx1_v.at[slot], sem_i.at[slot]).wait()

    def issue_fetch(t, b, slot):
        erow = (base0 + t * SUP) // 8
        pltpu.async_copy(e_hbm.at[pl.ds(erow, SUP // 8)], ev_v.at[b], sem_g.at[b])
        for j in range(NT):
            sl = pl.ds(j * K, K)
            pltpu.async_copy(p0_hbm.at[idx0_v.at[slot, j]], g0_v.at[b, sl], sem_g.at[b])
            pltpu.async_copy(p1_hbm.at[idx1_v.at[slot, j]], g1_v.at[b, sl], sem_g.at[b])

    def drain_fetch(t, b):
        erow = (base0 + t * SUP) // 8
        pltpu.make_async_copy(e_hbm.at[pl.ds(erow, SUP // 8)], ev_v.at[b], sem_g.at[b]).wait()
        pltpu.make_async_copy(p0_hbm.at[pl.ds(0, SUP)], g0_v.at[b], sem_g.at[b]).wait()
        pltpu.make_async_copy(p1_hbm.at[pl.ds(0, SUP)], g1_v.at[b], sem_g.at[b]).wait()

    def issue_scatter(b, slot):
        for j in range(NT):
            sl = pl.ds(j * K, K)
            pltpu.make_async_copy(m_v.at[b, sl], agg_sh.at[idx0_v.at[slot, j]],
                                  sem_s.at[b]).start(add=True)

    def drain_scatter(b, slot):
        for j in range(NT):
            sl = pl.ds(j * K, K)
            pltpu.make_async_copy(m_v.at[b, sl], agg_sh.at[idx0_v.at[slot, j]],
                                  sem_s.at[b]).wait()

    issue_idx(0, 0)
    issue_idx(1, 1)
    drain_idx(0, 0)
    issue_fetch(0, 0, 0)

    def step(t, carry):
        b = t % 2
        slot = t % 3

        @pl.when(t >= 1)
        def _():
            drain_scatter(1 - b, (t - 1) % 3)

        @pl.when(t + 2 < T_STEPS)
        def _():
            issue_idx(t + 2, (t + 2) % 3)

        @pl.when(t + 1 < T_STEPS)
        def _():
            drain_idx(t + 1, (t + 1) % 3)
            issue_fetch(t + 1, 1 - b, (t + 1) % 3)

        drain_fetch(t, b)

        def row4(u, carry2):
            for k in range(4):
                r = u * 4 + k
                er = r >> 3
                ec = (r & 7) * MSG
                for h in range(MSG // 16):
                    sl = pl.ds(h * 16, 16)
                    v = g0_v[b, r, sl] + g1_v[b, r, sl] + ev_v[b, er, pl.ds(ec + h * 16, 16)]
                    m_v[b, r, sl] = jnp.maximum(v, 0.0)
            return carry2

        lax.fori_loop(0, SUP // 4, row4, 0)
        issue_scatter(b, slot)
        return carry

    lax.fori_loop(0, T_STEPS, step, 0)
    drain_scatter((T_STEPS - 1) % 2, (T_STEPS - 1) % 3)
    plsc.subcore_barrier()
    pltpu.sync_copy(agg_sh.at[pl.ds(s * NPS, NPS)],
                    out_hbm.at[c, pl.ds(s * NPS, NPS)])


def _sc_conv(p0, p1, e, n0, n1, zeros):
    mesh = plsc.VectorSubcoreMesh(core_axis_name="c", subcore_axis_name="s")
    f = pl.kernel(
        _sc_conv_body,
        out_type=jax.ShapeDtypeStruct((NC, NPAD, MSG), jnp.float32),
        mesh=mesh,
        scratch_types=[
            pltpu.VMEM_SHARED((NPAD, MSG), jnp.float32),
            pltpu.VMEM((3, NT, K), jnp.int32),
            pltpu.VMEM((3, NT, K), jnp.int32),
            pltpu.VMEM((2, SUP, MSG), jnp.float32),
            pltpu.VMEM((2, SUP, MSG), jnp.float32),
            pltpu.VMEM((2, SUP // 8, 8 * MSG), jnp.float32),
            pltpu.VMEM((2, SUP, MSG), jnp.float32),
            pltpu.SemaphoreType.DMA((3,)),
            pltpu.SemaphoreType.DMA((2,)),
            pltpu.SemaphoreType.DMA((2,)),
        ],
        compiler_params=pltpu.CompilerParams(use_tc_tiling_on_sc=False),
    )
    return f(p0, p1, e, n0, n1, zeros)


_UBLK = 2000


def _node_up_body(x_ref, agg_ref, wna_ref, wnb_ref, bn_ref, wa2_ref, wb2_ref,
                  h_ref, p0_ref, p1_ref):
    aggs = agg_ref[0] + agg_ref[1]
    h = jnp.dot(x_ref[...], wna_ref[...], preferred_element_type=jnp.float32)
    h += jnp.dot(aggs, wnb_ref[...], preferred_element_type=jnp.float32)
    h = jnp.maximum(h + bn_ref[...], 0.0)
    h_ref[...] = h
    p0_ref[...] = jnp.dot(h, wa2_ref[...], preferred_element_type=jnp.float32)
    p1_ref[...] = jnp.dot(h, wb2_ref[...], preferred_element_type=jnp.float32)


def _node_update(x, agg, wnat, wnbt, bn, wa2t, wb2t):
    nblk = N_NODES // _UBLK
    return pl.pallas_call(
        _node_up_body,
        grid=(nblk,),
        in_specs=[
            pl.BlockSpec((_UBLK, D_FEAT), lambda i: (i, 0)),
            pl.BlockSpec((NC, _UBLK, MSG), lambda i: (0, i, 0)),
            pl.BlockSpec((D_FEAT, D_FEAT), lambda i: (0, 0)),
            pl.BlockSpec((MSG, D_FEAT), lambda i: (0, 0)),
            pl.BlockSpec((1, D_FEAT), lambda i: (0, 0)),
            pl.BlockSpec((D_FEAT, MSG), lambda i: (0, 0)),
            pl.BlockSpec((D_FEAT, MSG), lambda i: (0, 0)),
        ],
        out_specs=[
            pl.BlockSpec((_UBLK, D_FEAT), lambda i: (i, 0)),
            pl.BlockSpec((_UBLK, MSG), lambda i: (i, 0)),
            pl.BlockSpec((_UBLK, MSG), lambda i: (i, 0)),
        ],
        out_shape=[
            jax.ShapeDtypeStruct((N_NODES, D_FEAT), jnp.float32),
            jax.ShapeDtypeStruct((N_NODES, MSG), jnp.float32),
            jax.ShapeDtypeStruct((N_NODES, MSG), jnp.float32),
        ],
    )(x, agg, wnat, wnbt, bn, wa2t, wb2t)


_PBLK = 2000


def _pool_body(x_ref, agg_ref, batch_ref, wna_ref, wnb_ref, bn_ref,
               wg1_ref, bg1_ref, wg2_ref, bg2_ref, z_ref, sum_acc, cnt_acc):
    i = pl.program_id(0)
    aggs = agg_ref[0] + agg_ref[1]
    h = jnp.dot(x_ref[...], wna_ref[...], preferred_element_type=jnp.float32)
    h += jnp.dot(aggs, wnb_ref[...], preferred_element_type=jnp.float32)
    h = jnp.maximum(h + bn_ref[...], 0.0)

    b = batch_ref[0, 0, :]
    gids = lax.broadcasted_iota(jnp.int32, (N_GRAPHS, _PBLK), 0)
    onehot = (gids == b[None, :]).astype(jnp.float32)

    @pl.when(i == 0)
    def _init():
        sum_acc[...] = jnp.zeros_like(sum_acc)
        cnt_acc[...] = jnp.zeros_like(cnt_acc)

    sum_acc[...] += jnp.dot(onehot, h, preferred_element_type=jnp.float32)
    cnt_acc[...] += jnp.sum(onehot, axis=1, keepdims=True)

    @pl.when(i == pl.num_programs(0) - 1)
    def _final():
        means = sum_acc[...] / jnp.clip(cnt_acc[...], 1.0, None)
        g = jnp.dot(means, wg1_ref[...], preferred_element_type=jnp.float32)
        g = jnp.maximum(g + bg1_ref[...], 0.0)
        z = jnp.dot(g, wg2_ref[...], preferred_element_type=jnp.float32)
        z_ref[...] = z + bg2_ref[...]


def _pool_mlp(x, agg, batch3d, wnat, wnbt, bn, wg1t, bg1, wg2t, bg2):
    nblk = N_NODES // _PBLK
    return pl.pallas_call(
        _pool_body,
        grid=(nblk,),
        in_specs=[
            pl.BlockSpec((_PBLK, D_FEAT), lambda i: (i, 0)),
            pl.BlockSpec((NC, _PBLK, MSG), lambda i: (0, i, 0)),
            pl.BlockSpec((1, 1, _PBLK), lambda i: (i, 0, 0)),
            pl.BlockSpec((D_FEAT, D_FEAT), lambda i: (0, 0)),
            pl.BlockSpec((MSG, D_FEAT), lambda i: (0, 0)),
            pl.BlockSpec((1, D_FEAT), lambda i: (0, 0)),
            pl.BlockSpec((D_FEAT, HID), lambda i: (0, 0)),
            pl.BlockSpec((1, HID), lambda i: (0, 0)),
            pl.BlockSpec((HID, OUT), lambda i: (0, 0)),
            pl.BlockSpec((1, OUT), lambda i: (0, 0)),
        ],
        out_specs=pl.BlockSpec((N_GRAPHS, OUT), lambda i: (0, 0)),
        out_shape=jax.ShapeDtypeStruct((N_GRAPHS, OUT), jnp.float32),
        scratch_shapes=[
            pltpu.VMEM((N_GRAPHS, D_FEAT), jnp.float32),
            pltpu.VMEM((N_GRAPHS, 1), jnp.float32),
        ],
    )(x, agg, batch3d, wnat, wnbt, bn, wg1t, bg1, wg2t, bg2)


def kernel(x, edge_index, edge_attr, batch,
           We1, be1, Wn1, bn1, We2, be2, Wn2, bn2, Wg1, bg1, Wg2, bg2):
    n0 = edge_index[0].reshape(ROWS2, K)
    n1 = edge_index[1].reshape(ROWS2, K)

    wa1t = We1[:, :D_FEAT].T
    wb1t = We1[:, D_FEAT:2 * D_FEAT].T
    wc1t = We1[:, 2 * D_FEAT:].T
    wa2t = We2[:, :D_FEAT].T
    wb2t = We2[:, D_FEAT:2 * D_FEAT].T
    wc2t = We2[:, 2 * D_FEAT:].T
    wna1t = Wn1[:, :D_FEAT].T
    wnb1t = Wn1[:, D_FEAT:].T
    wna2t = Wn2[:, :D_FEAT].T
    wnb2t = Wn2[:, D_FEAT:].T
    wg1t = Wg1.T
    wg2t = Wg2.T

    w1bd = jnp.kron(jnp.eye(8, dtype=jnp.float32), wc1t)
    w2bd = jnp.kron(jnp.eye(8, dtype=jnp.float32), wc2t)
    be1r = jnp.tile(be1, 8).reshape(1, 8 * MSG)
    be2r = jnp.tile(be2, 8).reshape(1, 8 * MSG)
    bn1r = bn1.reshape(1, D_FEAT)
    bn2r = bn2.reshape(1, D_FEAT)
    bg1r = bg1.reshape(1, HID)
    bg2r = bg2.reshape(1, OUT)

    zeros = jnp.zeros((NPAD, MSG), jnp.float32)
    batch3d = batch.reshape(N_NODES // _PBLK, 1, _PBLK)
    ea8 = edge_attr.reshape(N_EDGES // 8, 8 * D_EDGE)

    e1, e2 = _edge_pre(ea8, w1bd, be1r, w2bd, be2r)
    p0, p1 = _proj(x, wa1t, wb1t)
    agg1 = _sc_conv(p0, p1, e1, n0, n1, zeros)
    h1, q0, q1 = _node_update(x, agg1, wna1t, wnb1t, bn1r, wa2t, wb2t)

    agg2 = _sc_conv(q0, q1, e2, n0, n1, zeros)

    z = _pool_mlp(h1, agg2, batch3d, wna2t, wnb2t, bn2r, wg1t, bg1r, wg2t, bg2r)
    return z


# speedup vs baseline: 10.4315x; 1.2432x over previous
import functools

import jax
import jax.numpy as jnp
from jax import lax
from jax.experimental import pallas as pl
from jax.experimental.pallas import tpu as pltpu
from jax.experimental.pallas import tpu_sc as plsc

N_NODES = 10000
N_EDGES = 320000
D_FEAT = 128
D_EDGE = 16
MSG = 32
HID = 128
OUT = 16
N_GRAPHS = 64

NC = 2
NS = 16
NW = NC * NS
EPW = N_EDGES // NW
K = 80
NPAD = 10240
NPS = NPAD // NS


EROWS = N_EDGES // 8
_EBLK = 2000


def _edge_pre_body(ea_ref, w1_ref, b1_ref, w2_ref, b2_ref, e1_ref, e2_ref):
    ea = ea_ref[...]
    e1_ref[...] = jnp.dot(ea, w1_ref[...], preferred_element_type=jnp.float32) + b1_ref[...]
    e2_ref[...] = jnp.dot(ea, w2_ref[...], preferred_element_type=jnp.float32) + b2_ref[...]


def _edge_pre(ea8, w1bd, b1, w2bd, b2):
    nblk = EROWS // _EBLK
    return pl.pallas_call(
        _edge_pre_body,
        grid=(nblk,),
        in_specs=[
            pl.BlockSpec((_EBLK, 128), lambda i: (i, 0)),
            pl.BlockSpec((128, 8 * MSG), lambda i: (0, 0)),
            pl.BlockSpec((1, 8 * MSG), lambda i: (0, 0)),
            pl.BlockSpec((128, 8 * MSG), lambda i: (0, 0)),
            pl.BlockSpec((1, 8 * MSG), lambda i: (0, 0)),
        ],
        out_specs=[
            pl.BlockSpec((_EBLK, 8 * MSG), lambda i: (i, 0)),
            pl.BlockSpec((_EBLK, 8 * MSG), lambda i: (i, 0)),
        ],
        out_shape=[
            jax.ShapeDtypeStruct((EROWS, 8 * MSG), jnp.float32),
            jax.ShapeDtypeStruct((EROWS, 8 * MSG), jnp.float32),
        ],
    )(ea8, w1bd, b1, w2bd, b2)


_NBLK = 2000


def _proj_body(x_ref, wa_ref, wb_ref, p0_ref, p1_ref):
    xv = x_ref[...]
    p0_ref[...] = jnp.dot(xv, wa_ref[...], preferred_element_type=jnp.float32).astype(jnp.bfloat16)
    p1_ref[...] = jnp.dot(xv, wb_ref[...], preferred_element_type=jnp.float32).astype(jnp.bfloat16)


def _proj(x, wat, wbt):
    nblk = N_NODES // _NBLK
    return pl.pallas_call(
        _proj_body,
        grid=(nblk,),
        in_specs=[
            pl.BlockSpec((_NBLK, D_FEAT), lambda i: (i, 0)),
            pl.BlockSpec((D_FEAT, MSG), lambda i: (0, 0)),
            pl.BlockSpec((D_FEAT, MSG), lambda i: (0, 0)),
        ],
        out_specs=[
            pl.BlockSpec((_NBLK, MSG), lambda i: (i, 0)),
            pl.BlockSpec((_NBLK, MSG), lambda i: (i, 0)),
        ],
        out_shape=[
            jax.ShapeDtypeStruct((N_NODES, MSG), jnp.bfloat16),
            jax.ShapeDtypeStruct((N_NODES, MSG), jnp.bfloat16),
        ],
    )(x, wat, wbt)


SUP = 400
NT = SUP // K
T_STEPS = EPW // SUP
ROWS2 = N_EDGES // K


def _sc_conv_body(p0_hbm, p1_hbm, e_hbm, n0_hbm, n1_hbm, z_hbm, out_hbm,
                  agg_sh, idx0_v, idx1_v, g0_v, g1_v, ev_v, m_v,
                  sem_i, sem_g, sem_s):
    c = lax.axis_index("c")
    s = lax.axis_index("s")
    wid = c * NS + s

    pltpu.sync_copy(z_hbm.at[pl.ds(s * NPS, NPS)], agg_sh.at[pl.ds(s * NPS, NPS)])
    plsc.subcore_barrier()

    irow0 = wid * (EPW // K)
    base0 = wid * EPW

    def issue_idx(t, slot):
        r = irow0 + t * NT
        pltpu.async_copy(n0_hbm.at[pl.ds(r, NT)], idx0_v.at[slot], sem_i.at[slot])
        pltpu.async_copy(n1_hbm.at[pl.ds(r, NT)], idx1_v.at[slot], sem_i.at[slot])

    def drain_idx(t, slot):
        r = irow0 + t * NT
        pltpu.make_async_copy(n0_hbm.at[pl.ds(r, NT)], idx0_v.at[slot], sem_i.at[slot]).wait()
        pltpu.make_async_copy(n1_hbm.at[pl.ds(r, NT)], idx1_v.at[slot], sem_i.at[slot]).wait()

    def issue_fetch(t, b, slot):
        erow = (base0 + t * SUP) // 8
        pltpu.async_copy(e_hbm.at[pl.ds(erow, SUP // 8)], ev_v.at[b], sem_g.at[b])
        for j in range(NT):
            sl = pl.ds(j * K, K)
            pltpu.async_copy(p0_hbm.at[idx0_v.at[slot, j]], g0_v.at[b, sl], sem_g.at[b])
            pltpu.async_copy(p1_hbm.at[idx1_v.at[slot, j]], g1_v.at[b, sl], sem_g.at[b])

    def drain_fetch(t, b):
        erow = (base0 + t * SUP) // 8
        pltpu.make_async_copy(e_hbm.at[pl.ds(erow, SUP // 8)], ev_v.at[b], sem_g.at[b]).wait()
        pltpu.make_async_copy(p0_hbm.at[pl.ds(0, SUP)], g0_v.at[b], sem_g.at[b]).wait()
        pltpu.make_async_copy(p1_hbm.at[pl.ds(0, SUP)], g1_v.at[b], sem_g.at[b]).wait()

    def issue_scatter(b, slot):
        for j in range(NT):
            sl = pl.ds(j * K, K)
            pltpu.make_async_copy(m_v.at[b, sl], agg_sh.at[idx0_v.at[slot, j]],
                                  sem_s.at[b]).start(add=True)

    def drain_scatter(b, slot):
        for j in range(NT):
            sl = pl.ds(j * K, K)
            pltpu.make_async_copy(m_v.at[b, sl], agg_sh.at[idx0_v.at[slot, j]],
                                  sem_s.at[b]).wait()

    issue_idx(0, 0)
    issue_idx(1, 1)
    drain_idx(0, 0)
    issue_fetch(0, 0, 0)

    def step(t, carry):
        b = t % 2
        slot = t % 3

        @pl.when(t >= 1)
        def _():
            drain_scatter(1 - b, (t - 1) % 3)

        @pl.when(t + 2 < T_STEPS)
        def _():
            issue_idx(t + 2, (t + 2) % 3)

        @pl.when(t + 1 < T_STEPS)
        def _():
            drain_idx(t + 1, (t + 1) % 3)
            issue_fetch(t + 1, 1 - b, (t + 1) % 3)

        drain_fetch(t, b)

        def row4(u, carry2):
            for k in range(4):
                r = u * 4 + k
                er = r >> 3
                ec = (r & 7) * MSG
                x0a, x0b = plsc.unpack(g0_v[b, r, :], format=plsc.PackFormat.INTERLEAVED)
                x1a, x1b = plsc.unpack(g1_v[b, r, :], format=plsc.PackFormat.INTERLEAVED)
                ea_ = ev_v[b, er, pl.ds(ec, 16)]
                eb_ = ev_v[b, er, pl.ds(ec + 16, 16)]
                m_v[b, r, pl.ds(0, 16)] = jnp.maximum(x0a + x1a + ea_, 0.0)
                m_v[b, r, pl.ds(16, 16)] = jnp.maximum(x0b + x1b + eb_, 0.0)
            return carry2

        lax.fori_loop(0, SUP // 4, row4, 0)
        issue_scatter(b, slot)
        return carry

    lax.fori_loop(0, T_STEPS, step, 0)
    drain_scatter((T_STEPS - 1) % 2, (T_STEPS - 1) % 3)
    plsc.subcore_barrier()
    pltpu.sync_copy(agg_sh.at[pl.ds(s * NPS, NPS)],
                    out_hbm.at[c, pl.ds(s * NPS, NPS)])


def _sc_conv(p0, p1, e, n0, n1, zeros):
    mesh = plsc.VectorSubcoreMesh(core_axis_name="c", subcore_axis_name="s")
    f = pl.kernel(
        _sc_conv_body,
        out_type=jax.ShapeDtypeStruct((NC, NPAD, MSG), jnp.float32),
        mesh=mesh,
        scratch_types=[
            pltpu.VMEM_SHARED((NPAD, MSG), jnp.float32),
            pltpu.VMEM((3, NT, K), jnp.int32),
            pltpu.VMEM((3, NT, K), jnp.int32),
            pltpu.VMEM((2, SUP, MSG), jnp.bfloat16),
            pltpu.VMEM((2, SUP, MSG), jnp.bfloat16),
            pltpu.VMEM((2, SUP // 8, 8 * MSG), jnp.float32),
            pltpu.VMEM((2, SUP, MSG), jnp.float32),
            pltpu.SemaphoreType.DMA((3,)),
            pltpu.SemaphoreType.DMA((2,)),
            pltpu.SemaphoreType.DMA((2,)),
        ],
        compiler_params=pltpu.CompilerParams(use_tc_tiling_on_sc=False,
                                             needs_layout_passes=False),
    )
    return f(p0, p1, e, n0, n1, zeros)


_UBLK = 2000


def _node_up_body(x_ref, agg_ref, wna_ref, wnb_ref, bn_ref, wa2_ref, wb2_ref,
                  h_ref, p0_ref, p1_ref):
    aggs = agg_ref[0] + agg_ref[1]
    h = jnp.dot(x_ref[...], wna_ref[...], preferred_element_type=jnp.float32)
    h += jnp.dot(aggs, wnb_ref[...], preferred_element_type=jnp.float32)
    h = jnp.maximum(h + bn_ref[...], 0.0)
    h_ref[...] = h
    p0_ref[...] = jnp.dot(h, wa2_ref[...], preferred_element_type=jnp.float32).astype(jnp.bfloat16)
    p1_ref[...] = jnp.dot(h, wb2_ref[...], preferred_element_type=jnp.float32).astype(jnp.bfloat16)


def _node_update(x, agg, wnat, wnbt, bn, wa2t, wb2t):
    nblk = N_NODES // _UBLK
    return pl.pallas_call(
        _node_up_body,
        grid=(nblk,),
        in_specs=[
            pl.BlockSpec((_UBLK, D_FEAT), lambda i: (i, 0)),
            pl.BlockSpec((NC, _UBLK, MSG), lambda i: (0, i, 0)),
            pl.BlockSpec((D_FEAT, D_FEAT), lambda i: (0, 0)),
            pl.BlockSpec((MSG, D_FEAT), lambda i: (0, 0)),
            pl.BlockSpec((1, D_FEAT), lambda i: (0, 0)),
            pl.BlockSpec((D_FEAT, MSG), lambda i: (0, 0)),
            pl.BlockSpec((D_FEAT, MSG), lambda i: (0, 0)),
        ],
        out_specs=[
            pl.BlockSpec((_UBLK, D_FEAT), lambda i: (i, 0)),
            pl.BlockSpec((_UBLK, MSG), lambda i: (i, 0)),
            pl.BlockSpec((_UBLK, MSG), lambda i: (i, 0)),
        ],
        out_shape=[
            jax.ShapeDtypeStruct((N_NODES, D_FEAT), jnp.float32),
            jax.ShapeDtypeStruct((N_NODES, MSG), jnp.bfloat16),
            jax.ShapeDtypeStruct((N_NODES, MSG), jnp.bfloat16),
        ],
    )(x, agg, wnat, wnbt, bn, wa2t, wb2t)


_PBLK = 2000


def _pool_body(x_ref, agg_ref, batch_ref, wna_ref, wnb_ref, bn_ref,
               wg1_ref, bg1_ref, wg2_ref, bg2_ref, z_ref, sum_acc, cnt_acc):
    i = pl.program_id(0)
    aggs = agg_ref[0] + agg_ref[1]
    h = jnp.dot(x_ref[...], wna_ref[...], preferred_element_type=jnp.float32)
    h += jnp.dot(aggs, wnb_ref[...], preferred_element_type=jnp.float32)
    h = jnp.maximum(h + bn_ref[...], 0.0)

    b = batch_ref[0, 0, :]
    gids = lax.broadcasted_iota(jnp.int32, (N_GRAPHS, _PBLK), 0)
    onehot = (gids == b[None, :]).astype(jnp.float32)

    @pl.when(i == 0)
    def _init():
        sum_acc[...] = jnp.zeros_like(sum_acc)
        cnt_acc[...] = jnp.zeros_like(cnt_acc)

    sum_acc[...] += jnp.dot(onehot, h, preferred_element_type=jnp.float32)
    cnt_acc[...] += jnp.sum(onehot, axis=1, keepdims=True)

    @pl.when(i == pl.num_programs(0) - 1)
    def _final():
        means = sum_acc[...] / jnp.clip(cnt_acc[...], 1.0, None)
        g = jnp.dot(means, wg1_ref[...], preferred_element_type=jnp.float32)
        g = jnp.maximum(g + bg1_ref[...], 0.0)
        z = jnp.dot(g, wg2_ref[...], preferred_element_type=jnp.float32)
        z_ref[...] = z + bg2_ref[...]


def _pool_mlp(x, agg, batch3d, wnat, wnbt, bn, wg1t, bg1, wg2t, bg2):
    nblk = N_NODES // _PBLK
    return pl.pallas_call(
        _pool_body,
        grid=(nblk,),
        in_specs=[
            pl.BlockSpec((_PBLK, D_FEAT), lambda i: (i, 0)),
            pl.BlockSpec((NC, _PBLK, MSG), lambda i: (0, i, 0)),
            pl.BlockSpec((1, 1, _PBLK), lambda i: (i, 0, 0)),
            pl.BlockSpec((D_FEAT, D_FEAT), lambda i: (0, 0)),
            pl.BlockSpec((MSG, D_FEAT), lambda i: (0, 0)),
            pl.BlockSpec((1, D_FEAT), lambda i: (0, 0)),
            pl.BlockSpec((D_FEAT, HID), lambda i: (0, 0)),
            pl.BlockSpec((1, HID), lambda i: (0, 0)),
            pl.BlockSpec((HID, OUT), lambda i: (0, 0)),
            pl.BlockSpec((1, OUT), lambda i: (0, 0)),
        ],
        out_specs=pl.BlockSpec((N_GRAPHS, OUT), lambda i: (0, 0)),
        out_shape=jax.ShapeDtypeStruct((N_GRAPHS, OUT), jnp.float32),
        scratch_shapes=[
            pltpu.VMEM((N_GRAPHS, D_FEAT), jnp.float32),
            pltpu.VMEM((N_GRAPHS, 1), jnp.float32),
        ],
    )(x, agg, batch3d, wnat, wnbt, bn, wg1t, bg1, wg2t, bg2)


def kernel(x, edge_index, edge_attr, batch,
           We1, be1, Wn1, bn1, We2, be2, Wn2, bn2, Wg1, bg1, Wg2, bg2):
    n0 = edge_index[0].reshape(ROWS2, K)
    n1 = edge_index[1].reshape(ROWS2, K)

    wa1t = We1[:, :D_FEAT].T
    wb1t = We1[:, D_FEAT:2 * D_FEAT].T
    wc1t = We1[:, 2 * D_FEAT:].T
    wa2t = We2[:, :D_FEAT].T
    wb2t = We2[:, D_FEAT:2 * D_FEAT].T
    wc2t = We2[:, 2 * D_FEAT:].T
    wna1t = Wn1[:, :D_FEAT].T
    wnb1t = Wn1[:, D_FEAT:].T
    wna2t = Wn2[:, :D_FEAT].T
    wnb2t = Wn2[:, D_FEAT:].T
    wg1t = Wg1.T
    wg2t = Wg2.T

    perm = jnp.concatenate([jnp.arange(0, MSG, 2), jnp.arange(1, MSG, 2)])
    wc1tp = wc1t[:, perm]
    wc2tp = wc2t[:, perm]
    wnb1tp = wnb1t[perm, :]
    wnb2tp = wnb2t[perm, :]
    w1bd = jnp.kron(jnp.eye(8, dtype=jnp.float32), wc1tp)
    w2bd = jnp.kron(jnp.eye(8, dtype=jnp.float32), wc2tp)
    be1r = jnp.tile(be1[perm], 8).reshape(1, 8 * MSG)
    be2r = jnp.tile(be2[perm], 8).reshape(1, 8 * MSG)
    bn1r = bn1.reshape(1, D_FEAT)
    bn2r = bn2.reshape(1, D_FEAT)
    bg1r = bg1.reshape(1, HID)
    bg2r = bg2.reshape(1, OUT)

    zeros = jnp.zeros((NPAD, MSG), jnp.float32)
    batch3d = batch.reshape(N_NODES // _PBLK, 1, _PBLK)
    ea8 = edge_attr.reshape(N_EDGES // 8, 8 * D_EDGE)

    e1, e2 = _edge_pre(ea8, w1bd, be1r, w2bd, be2r)
    p0, p1 = _proj(x, wa1t, wb1t)
    agg1 = _sc_conv(p0, p1, e1, n0, n1, zeros)
    h1, q0, q1 = _node_update(x, agg1, wna1t, wnb1tp, bn1r, wa2t, wb2t)

    agg2 = _sc_conv(q0, q1, e2, n0, n1, zeros)

    z = _pool_mlp(h1, agg2, batch3d, wna2t, wnb2tp, bn2r, wg1t, bg1r, wg2t, bg2r)
    return z


# speedup vs baseline: 11.7042x; 1.1220x over previous
import functools

import jax
import jax.numpy as jnp
from jax import lax
from jax.experimental import pallas as pl
from jax.experimental.pallas import tpu as pltpu
from jax.experimental.pallas import tpu_sc as plsc

N_NODES = 10000
N_EDGES = 320000
D_FEAT = 128
D_EDGE = 16
MSG = 32
HID = 128
OUT = 16
N_GRAPHS = 64

NC = 2
NS = 16
NW = NC * NS
EPW = N_EDGES // NW
K = 80
NPAD = 10240
NPS = NPAD // NS


EROWS = N_EDGES // 8
_EBLK = 2000


def _pack_pair(lo, hi):
    lo16 = lax.bitcast_convert_type(lo.astype(jnp.bfloat16), jnp.uint16).astype(jnp.uint32)
    hi16 = lax.bitcast_convert_type(hi.astype(jnp.bfloat16), jnp.uint16).astype(jnp.uint32)
    return lax.bitcast_convert_type((hi16 << 16) | lo16, jnp.float32)


def _edge_pre_body(ea_ref, w1l_ref, b1l_ref, w1h_ref, b1h_ref,
                   w2l_ref, b2l_ref, w2h_ref, b2h_ref, e1_ref, e2_ref):
    ea = ea_ref[...]

    def half(w_ref, b_ref):
        return jnp.dot(ea, w_ref[...], preferred_element_type=jnp.float32) + b_ref[...]

    e1_ref[...] = _pack_pair(half(w1l_ref, b1l_ref), half(w1h_ref, b1h_ref))
    e2_ref[...] = _pack_pair(half(w2l_ref, b2l_ref), half(w2h_ref, b2h_ref))


def _edge_pre(ea8, *wb):
    nblk = EROWS // _EBLK
    wspec = pl.BlockSpec((128, 128), lambda i: (0, 0))
    bspec = pl.BlockSpec((1, 128), lambda i: (0, 0))
    return pl.pallas_call(
        _edge_pre_body,
        grid=(nblk,),
        in_specs=[pl.BlockSpec((_EBLK, 128), lambda i: (i, 0))]
                 + [wspec, bspec] * 4,
        out_specs=[
            pl.BlockSpec((_EBLK, 128), lambda i: (i, 0)),
            pl.BlockSpec((_EBLK, 128), lambda i: (i, 0)),
        ],
        out_shape=[
            jax.ShapeDtypeStruct((EROWS, 128), jnp.float32),
            jax.ShapeDtypeStruct((EROWS, 128), jnp.float32),
        ],
    )(ea8, *wb)


_NBLK = 2000


def _proj_body(x_ref, wa_ref, wb_ref, p0_ref, p1_ref):
    xv = x_ref[...]
    p0_ref[...] = jnp.dot(xv, wa_ref[...], preferred_element_type=jnp.float32).astype(jnp.bfloat16)
    p1_ref[...] = jnp.dot(xv, wb_ref[...], preferred_element_type=jnp.float32).astype(jnp.bfloat16)


def _proj(x, wat, wbt):
    nblk = N_NODES // _NBLK
    return pl.pallas_call(
        _proj_body,
        grid=(nblk,),
        in_specs=[
            pl.BlockSpec((_NBLK, D_FEAT), lambda i: (i, 0)),
            pl.BlockSpec((D_FEAT, MSG), lambda i: (0, 0)),
            pl.BlockSpec((D_FEAT, MSG), lambda i: (0, 0)),
        ],
        out_specs=[
            pl.BlockSpec((_NBLK, MSG), lambda i: (i, 0)),
            pl.BlockSpec((_NBLK, MSG), lambda i: (i, 0)),
        ],
        out_shape=[
            jax.ShapeDtypeStruct((N_NODES, MSG), jnp.bfloat16),
            jax.ShapeDtypeStruct((N_NODES, MSG), jnp.bfloat16),
        ],
    )(x, wat, wbt)


SUP = 400
NT = SUP // K
T_STEPS = EPW // SUP
ROWS2 = N_EDGES // K


def _sc_conv_body(p0_hbm, p1_hbm, e_hbm, ei_hbm, z_hbm, out_hbm,
                  agg_sh, idx0_v, idx1_v, g0_v, g1_v, ev_v, m_v,
                  sem_i, sem_g, sem_s):
    c = lax.axis_index("c")
    s = lax.axis_index("s")
    wid = c * NS + s

    pltpu.sync_copy(z_hbm.at[pl.ds(s * NPS, NPS)], agg_sh.at[pl.ds(s * NPS, NPS)])
    plsc.subcore_barrier()

    irow0 = wid * (EPW // K)
    base0 = wid * EPW

    def issue_idx(t, slot):
        r = irow0 + t * NT
        pltpu.async_copy(ei_hbm.at[0, pl.ds(r, NT)], idx0_v.at[slot], sem_i.at[slot])
        pltpu.async_copy(ei_hbm.at[1, pl.ds(r, NT)], idx1_v.at[slot], sem_i.at[slot])

    def drain_idx(t, slot):
        r = irow0 + t * NT
        pltpu.make_async_copy(ei_hbm.at[0, pl.ds(r, NT)], idx0_v.at[slot], sem_i.at[slot]).wait()
        pltpu.make_async_copy(ei_hbm.at[1, pl.ds(r, NT)], idx1_v.at[slot], sem_i.at[slot]).wait()

    def issue_fetch(t, b, slot):
        erow = (base0 + t * SUP) // 8
        pltpu.async_copy(e_hbm.at[pl.ds(erow, SUP // 8)], ev_v.at[b], sem_g.at[b])
        for j in range(NT):
            sl = pl.ds(j * K, K)
            pltpu.async_copy(p0_hbm.at[idx0_v.at[slot, j]], g0_v.at[b, sl], sem_g.at[b])
            pltpu.async_copy(p1_hbm.at[idx1_v.at[slot, j]], g1_v.at[b, sl], sem_g.at[b])

    def drain_fetch(t, b):
        erow = (base0 + t * SUP) // 8
        pltpu.make_async_copy(e_hbm.at[pl.ds(erow, SUP // 8)], ev_v.at[b], sem_g.at[b]).wait()
        pltpu.make_async_copy(p0_hbm.at[pl.ds(0, SUP)], g0_v.at[b], sem_g.at[b]).wait()
        pltpu.make_async_copy(p1_hbm.at[pl.ds(0, SUP)], g1_v.at[b], sem_g.at[b]).wait()

    def issue_scatter(b, slot):
        for j in range(NT):
            sl = pl.ds(j * K, K)
            pltpu.make_async_copy(m_v.at[b, sl], agg_sh.at[idx0_v.at[slot, j]],
                                  sem_s.at[b]).start(add=True)

    def drain_scatter(b, slot):
        for j in range(NT):
            sl = pl.ds(j * K, K)
            pltpu.make_async_copy(m_v.at[b, sl], agg_sh.at[idx0_v.at[slot, j]],
                                  sem_s.at[b]).wait()

    issue_idx(0, 0)
    issue_idx(1, 1)
    drain_idx(0, 0)
    issue_fetch(0, 0, 0)

    def step(t, carry):
        b = t % 2
        slot = t % 3

        @pl.when(t >= 1)
        def _():
            drain_scatter(1 - b, (t - 1) % 3)

        @pl.when(t + 2 < T_STEPS)
        def _():
            issue_idx(t + 2, (t + 2) % 3)

        @pl.when(t + 1 < T_STEPS)
        def _():
            drain_idx(t + 1, (t + 1) % 3)
            issue_fetch(t + 1, 1 - b, (t + 1) % 3)

        drain_fetch(t, b)

        def row4(u, carry2):
            for k in range(4):
                r = u * 4 + k
                er = r >> 3
                ec = (r & 7) * 16
                x0a, x0b = plsc.unpack(g0_v[b, r, :], format=plsc.PackFormat.INTERLEAVED)
                x1a, x1b = plsc.unpack(g1_v[b, r, :], format=plsc.PackFormat.INTERLEAVED)
                ew = plsc.bitcast(ev_v[b, er, pl.ds(ec, 16)], jnp.bfloat16)
                ea_, eb_ = plsc.unpack(ew, format=plsc.PackFormat.INTERLEAVED)
                m_v[b, r, pl.ds(0, 16)] = jnp.maximum(x0a + x1a + ea_, 0.0)
                m_v[b, r, pl.ds(16, 16)] = jnp.maximum(x0b + x1b + eb_, 0.0)
            return carry2

        lax.fori_loop(0, SUP // 4, row4, 0)
        issue_scatter(b, slot)
        return carry

    lax.fori_loop(0, T_STEPS, step, 0)
    drain_scatter((T_STEPS - 1) % 2, (T_STEPS - 1) % 3)
    plsc.subcore_barrier()
    pltpu.sync_copy(agg_sh.at[pl.ds(s * NPS, NPS)],
                    out_hbm.at[c, pl.ds(s * NPS, NPS)])


def _sc_conv(p0, p1, e, ei3, zeros):
    mesh = plsc.VectorSubcoreMesh(core_axis_name="c", subcore_axis_name="s")
    f = pl.kernel(
        _sc_conv_body,
        out_type=jax.ShapeDtypeStruct((NC, NPAD, MSG), jnp.float32),
        mesh=mesh,
        scratch_types=[
            pltpu.VMEM_SHARED((NPAD, MSG), jnp.float32),
            pltpu.VMEM((3, NT, K), jnp.int32),
            pltpu.VMEM((3, NT, K), jnp.int32),
            pltpu.VMEM((2, SUP, MSG), jnp.bfloat16),
            pltpu.VMEM((2, SUP, MSG), jnp.bfloat16),
            pltpu.VMEM((2, SUP // 8, 128), jnp.float32),
            pltpu.VMEM((2, SUP, MSG), jnp.float32),
            pltpu.SemaphoreType.DMA((3,)),
            pltpu.SemaphoreType.DMA((2,)),
            pltpu.SemaphoreType.DMA((2,)),
        ],
        compiler_params=pltpu.CompilerParams(use_tc_tiling_on_sc=False,
                                             needs_layout_passes=False),
    )
    return f(p0, p1, e, ei3, zeros)


_UBLK = 2000


def _node_up_body(x_ref, agg_ref, wna_ref, wnb_ref, bn_ref, wa2_ref, wb2_ref,
                  h_ref, p0_ref, p1_ref):
    aggs = agg_ref[0] + agg_ref[1]
    h = jnp.dot(x_ref[...], wna_ref[...], preferred_element_type=jnp.float32)
    h += jnp.dot(aggs, wnb_ref[...], preferred_element_type=jnp.float32)
    h = jnp.maximum(h + bn_ref[...], 0.0)
    h_ref[...] = h
    p0_ref[...] = jnp.dot(h, wa2_ref[...], preferred_element_type=jnp.float32).astype(jnp.bfloat16)
    p1_ref[...] = jnp.dot(h, wb2_ref[...], preferred_element_type=jnp.float32).astype(jnp.bfloat16)


def _node_update(x, agg, wnat, wnbt, bn, wa2t, wb2t):
    nblk = N_NODES // _UBLK
    return pl.pallas_call(
        _node_up_body,
        grid=(nblk,),
        in_specs=[
            pl.BlockSpec((_UBLK, D_FEAT), lambda i: (i, 0)),
            pl.BlockSpec((NC, _UBLK, MSG), lambda i: (0, i, 0)),
            pl.BlockSpec((D_FEAT, D_FEAT), lambda i: (0, 0)),
            pl.BlockSpec((MSG, D_FEAT), lambda i: (0, 0)),
            pl.BlockSpec((1, D_FEAT), lambda i: (0, 0)),
            pl.BlockSpec((D_FEAT, MSG), lambda i: (0, 0)),
            pl.BlockSpec((D_FEAT, MSG), lambda i: (0, 0)),
        ],
        out_specs=[
            pl.BlockSpec((_UBLK, D_FEAT), lambda i: (i, 0)),
            pl.BlockSpec((_UBLK, MSG), lambda i: (i, 0)),
            pl.BlockSpec((_UBLK, MSG), lambda i: (i, 0)),
        ],
        out_shape=[
            jax.ShapeDtypeStruct((N_NODES, D_FEAT), jnp.float32),
            jax.ShapeDtypeStruct((N_NODES, MSG), jnp.bfloat16),
            jax.ShapeDtypeStruct((N_NODES, MSG), jnp.bfloat16),
        ],
    )(x, agg, wnat, wnbt, bn, wa2t, wb2t)


_PBLK = 2000


def _pool_body(x_ref, agg_ref, batch_ref, wna_ref, wnb_ref, bn_ref,
               wg1_ref, bg1_ref, wg2_ref, bg2_ref, z_ref, sum_acc, cnt_acc):
    i = pl.program_id(0)
    aggs = agg_ref[0] + agg_ref[1]
    h = jnp.dot(x_ref[...], wna_ref[...], preferred_element_type=jnp.float32)
    h += jnp.dot(aggs, wnb_ref[...], preferred_element_type=jnp.float32)
    h = jnp.maximum(h + bn_ref[...], 0.0)

    b = batch_ref[0, 0, :]
    gids = lax.broadcasted_iota(jnp.int32, (N_GRAPHS, _PBLK), 0)
    onehot = (gids == b[None, :]).astype(jnp.float32)

    @pl.when(i == 0)
    def _init():
        sum_acc[...] = jnp.zeros_like(sum_acc)
        cnt_acc[...] = jnp.zeros_like(cnt_acc)

    sum_acc[...] += jnp.dot(onehot, h, preferred_element_type=jnp.float32)
    cnt_acc[...] += jnp.sum(onehot, axis=1, keepdims=True)

    @pl.when(i == pl.num_programs(0) - 1)
    def _final():
        means = sum_acc[...] / jnp.clip(cnt_acc[...], 1.0, None)
        g = jnp.dot(means, wg1_ref[...], preferred_element_type=jnp.float32)
        g = jnp.maximum(g + bg1_ref[...], 0.0)
        z = jnp.dot(g, wg2_ref[...], preferred_element_type=jnp.float32)
        z_ref[...] = z + bg2_ref[...]


def _pool_mlp(x, agg, batch3d, wnat, wnbt, bn, wg1t, bg1, wg2t, bg2):
    nblk = N_NODES // _PBLK
    return pl.pallas_call(
        _pool_body,
        grid=(nblk,),
        in_specs=[
            pl.BlockSpec((_PBLK, D_FEAT), lambda i: (i, 0)),
            pl.BlockSpec((NC, _PBLK, MSG), lambda i: (0, i, 0)),
            pl.BlockSpec((1, 1, _PBLK), lambda i: (i, 0, 0)),
            pl.BlockSpec((D_FEAT, D_FEAT), lambda i: (0, 0)),
            pl.BlockSpec((MSG, D_FEAT), lambda i: (0, 0)),
            pl.BlockSpec((1, D_FEAT), lambda i: (0, 0)),
            pl.BlockSpec((D_FEAT, HID), lambda i: (0, 0)),
            pl.BlockSpec((1, HID), lambda i: (0, 0)),
            pl.BlockSpec((HID, OUT), lambda i: (0, 0)),
            pl.BlockSpec((1, OUT), lambda i: (0, 0)),
        ],
        out_specs=pl.BlockSpec((N_GRAPHS, OUT), lambda i: (0, 0)),
        out_shape=jax.ShapeDtypeStruct((N_GRAPHS, OUT), jnp.float32),
        scratch_shapes=[
            pltpu.VMEM((N_GRAPHS, D_FEAT), jnp.float32),
            pltpu.VMEM((N_GRAPHS, 1), jnp.float32),
        ],
    )(x, agg, batch3d, wnat, wnbt, bn, wg1t, bg1, wg2t, bg2)


def kernel(x, edge_index, edge_attr, batch,
           We1, be1, Wn1, bn1, We2, be2, Wn2, bn2, Wg1, bg1, Wg2, bg2):
    ei3 = edge_index.reshape(2, ROWS2, K)

    wa1t = We1[:, :D_FEAT].T
    wb1t = We1[:, D_FEAT:2 * D_FEAT].T
    wc1t = We1[:, 2 * D_FEAT:].T
    wa2t = We2[:, :D_FEAT].T
    wb2t = We2[:, D_FEAT:2 * D_FEAT].T
    wc2t = We2[:, 2 * D_FEAT:].T
    wna1t = Wn1[:, :D_FEAT].T
    wnb1t = Wn1[:, D_FEAT:].T
    wna2t = Wn2[:, :D_FEAT].T
    wnb2t = Wn2[:, D_FEAT:].T
    wg1t = Wg1.T
    wg2t = Wg2.T

    perm = jnp.concatenate([jnp.arange(0, MSG, 2), jnp.arange(1, MSG, 2)])
    wnb1tp = wnb1t[perm, :]
    wnb2tp = wnb2t[perm, :]
    eye8 = jnp.eye(8, dtype=jnp.float32)
    w1lo = jnp.kron(eye8, wc1t[:, 0::2])
    w1hi = jnp.kron(eye8, wc1t[:, 1::2])
    w2lo = jnp.kron(eye8, wc2t[:, 0::2])
    w2hi = jnp.kron(eye8, wc2t[:, 1::2])
    b1lo = jnp.tile(be1[0::2], 8).reshape(1, 128)
    b1hi = jnp.tile(be1[1::2], 8).reshape(1, 128)
    b2lo = jnp.tile(be2[0::2], 8).reshape(1, 128)
    b2hi = jnp.tile(be2[1::2], 8).reshape(1, 128)
    bn1r = bn1.reshape(1, D_FEAT)
    bn2r = bn2.reshape(1, D_FEAT)
    bg1r = bg1.reshape(1, HID)
    bg2r = bg2.reshape(1, OUT)

    zeros = jnp.zeros((NPAD, MSG), jnp.float32)
    batch3d = batch.reshape(N_NODES // _PBLK, 1, _PBLK)
    ea8 = edge_attr.reshape(N_EDGES // 8, 8 * D_EDGE)

    e1, e2 = _edge_pre(ea8, w1lo, b1lo, w1hi, b1hi, w2lo, b2lo, w2hi, b2hi)
    p0, p1 = _proj(x, wa1t, wb1t)
    agg1 = _sc_conv(p0, p1, e1, ei3, zeros)
    h1, q0, q1 = _node_update(x, agg1, wna1t, wnb1tp, bn1r, wa2t, wb2t)

    agg2 = _sc_conv(q0, q1, e2, ei3, zeros)

    z = _pool_mlp(h1, agg2, batch3d, wna2t, wnb2tp, bn2r, wg1t, bg1r, wg2t, bg2r)
    return z


# speedup vs baseline: 11.7044x; 1.0000x over previous
import functools

import jax
import jax.numpy as jnp
from jax import lax
from jax.experimental import pallas as pl
from jax.experimental.pallas import tpu as pltpu
from jax.experimental.pallas import tpu_sc as plsc

N_NODES = 10000
N_EDGES = 320000
D_FEAT = 128
D_EDGE = 16
MSG = 32
HID = 128
OUT = 16
N_GRAPHS = 64

NC = 2
NS = 16
NW = NC * NS
EPW = N_EDGES // NW
K = 100
NPAD = 10240
NPS = NPAD // NS


EROWS = N_EDGES // 8
_EBLK = 2000


def _pack_pair(lo, hi):
    lo16 = lax.bitcast_convert_type(lo.astype(jnp.bfloat16), jnp.uint16).astype(jnp.uint32)
    hi16 = lax.bitcast_convert_type(hi.astype(jnp.bfloat16), jnp.uint16).astype(jnp.uint32)
    return lax.bitcast_convert_type((hi16 << 16) | lo16, jnp.float32)


def _edge_pre_body(ea_ref, x_ref, w1l_ref, b1l_ref, w1h_ref, b1h_ref,
                   w2l_ref, b2l_ref, w2h_ref, b2h_ref, wa_ref, wb_ref,
                   e1_ref, e2_ref, p0_ref, p1_ref):
    ea = ea_ref[...]

    def half(w_ref, b_ref):
        return jnp.dot(ea, w_ref[...], preferred_element_type=jnp.float32) + b_ref[...]

    e1_ref[...] = _pack_pair(half(w1l_ref, b1l_ref), half(w1h_ref, b1h_ref))
    e2_ref[...] = _pack_pair(half(w2l_ref, b2l_ref), half(w2h_ref, b2h_ref))

    @pl.when(pl.program_id(0) < N_NODES // _NBLK)
    def _():
        xv = x_ref[...]
        p0_ref[...] = jnp.dot(xv, wa_ref[...], preferred_element_type=jnp.float32).astype(jnp.bfloat16)
        p1_ref[...] = jnp.dot(xv, wb_ref[...], preferred_element_type=jnp.float32).astype(jnp.bfloat16)


def _edge_pre(ea8, x, *wb):
    nblk = EROWS // _EBLK
    wspec = pl.BlockSpec((128, 128), lambda i: (0, 0))
    bspec = pl.BlockSpec((1, 128), lambda i: (0, 0))
    nlast = N_NODES // _NBLK - 1
    return pl.pallas_call(
        _edge_pre_body,
        grid=(nblk,),
        in_specs=[pl.BlockSpec((_EBLK, 128), lambda i: (i, 0)),
                  pl.BlockSpec((_NBLK, D_FEAT), lambda i: (jnp.minimum(i, nlast), 0))]
                 + [wspec, bspec] * 4
                 + [pl.BlockSpec((D_FEAT, MSG), lambda i: (0, 0))] * 2,
        out_specs=[
            pl.BlockSpec((_EBLK, 128), lambda i: (i, 0)),
            pl.BlockSpec((_EBLK, 128), lambda i: (i, 0)),
            pl.BlockSpec((_NBLK, MSG), lambda i: (jnp.minimum(i, nlast), 0)),
            pl.BlockSpec((_NBLK, MSG), lambda i: (jnp.minimum(i, nlast), 0)),
        ],
        out_shape=[
            jax.ShapeDtypeStruct((EROWS, 128), jnp.float32),
            jax.ShapeDtypeStruct((EROWS, 128), jnp.float32),
            jax.ShapeDtypeStruct((N_NODES, MSG), jnp.bfloat16),
            jax.ShapeDtypeStruct((N_NODES, MSG), jnp.bfloat16),
        ],
    )(ea8, x, *wb)


_NBLK = 2000


SUP = 400
NT = SUP // K
T_STEPS = EPW // SUP
ROWS2 = N_EDGES // K


def _sc_conv_body(p0_hbm, p1_hbm, e_hbm, ei_hbm, z_hbm, out_hbm,
                  agg_sh, idx0_v, idx1_v, g0_v, g1_v, ev_v, m_v,
                  sem_i, sem_g, sem_s):
    c = lax.axis_index("c")
    s = lax.axis_index("s")
    wid = c * NS + s

    pltpu.sync_copy(z_hbm.at[pl.ds(s * NPS, NPS)], agg_sh.at[pl.ds(s * NPS, NPS)])
    plsc.subcore_barrier()

    irow0 = wid * (EPW // K)
    base0 = wid * EPW

    def issue_idx(t, slot):
        r = irow0 + t * NT
        pltpu.async_copy(ei_hbm.at[0, pl.ds(r, NT)], idx0_v.at[slot], sem_i.at[slot])
        pltpu.async_copy(ei_hbm.at[1, pl.ds(r, NT)], idx1_v.at[slot], sem_i.at[slot])

    def drain_idx(t, slot):
        r = irow0 + t * NT
        pltpu.make_async_copy(ei_hbm.at[0, pl.ds(r, NT)], idx0_v.at[slot], sem_i.at[slot]).wait()
        pltpu.make_async_copy(ei_hbm.at[1, pl.ds(r, NT)], idx1_v.at[slot], sem_i.at[slot]).wait()

    def issue_fetch(t, b, slot):
        erow = (base0 + t * SUP) // 8
        pltpu.async_copy(e_hbm.at[pl.ds(erow, SUP // 8)], ev_v.at[b], sem_g.at[b])
        for j in range(NT):
            sl = pl.ds(j * K, K)
            pltpu.async_copy(p0_hbm.at[idx0_v.at[slot, j]], g0_v.at[b, sl], sem_g.at[b])
            pltpu.async_copy(p1_hbm.at[idx1_v.at[slot, j]], g1_v.at[b, sl], sem_g.at[b])

    def drain_fetch(t, b):
        erow = (base0 + t * SUP) // 8
        pltpu.make_async_copy(e_hbm.at[pl.ds(erow, SUP // 8)], ev_v.at[b], sem_g.at[b]).wait()
        pltpu.make_async_copy(p0_hbm.at[pl.ds(0, SUP)], g0_v.at[b], sem_g.at[b]).wait()
        pltpu.make_async_copy(p1_hbm.at[pl.ds(0, SUP)], g1_v.at[b], sem_g.at[b]).wait()

    def issue_scatter(b, slot):
        for j in range(NT):
            sl = pl.ds(j * K, K)
            pltpu.make_async_copy(m_v.at[b, sl], agg_sh.at[idx0_v.at[slot, j]],
                                  sem_s.at[b]).start(add=True)

    def drain_scatter(b, slot):
        for j in range(NT):
            sl = pl.ds(j * K, K)
            pltpu.make_async_copy(m_v.at[b, sl], agg_sh.at[idx0_v.at[slot, j]],
                                  sem_s.at[b]).wait()

    issue_idx(0, 0)
    issue_idx(1, 1)
    drain_idx(0, 0)
    issue_fetch(0, 0, 0)

    def step(t, carry):
        b = t % 2
        slot = t % 3

        @pl.when(t >= 1)
        def _():
            drain_scatter(1 - b, (t - 1) % 3)

        @pl.when(t + 2 < T_STEPS)
        def _():
            issue_idx(t + 2, (t + 2) % 3)

        @pl.when(t + 1 < T_STEPS)
        def _():
            drain_idx(t + 1, (t + 1) % 3)
            issue_fetch(t + 1, 1 - b, (t + 1) % 3)

        drain_fetch(t, b)

        def row4(u, carry2):
            for k in range(4):
                r = u * 4 + k
                er = r >> 3
                ec = (r & 7) * 16
                x0a, x0b = plsc.unpack(g0_v[b, r, :], format=plsc.PackFormat.INTERLEAVED)
                x1a, x1b = plsc.unpack(g1_v[b, r, :], format=plsc.PackFormat.INTERLEAVED)
                ew = plsc.bitcast(ev_v[b, er, pl.ds(ec, 16)], jnp.bfloat16)
                ea_, eb_ = plsc.unpack(ew, format=plsc.PackFormat.INTERLEAVED)
                m_v[b, r, pl.ds(0, 16)] = jnp.maximum(x0a + x1a + ea_, 0.0)
                m_v[b, r, pl.ds(16, 16)] = jnp.maximum(x0b + x1b + eb_, 0.0)
            return carry2

        lax.fori_loop(0, SUP // 4, row4, 0)
        issue_scatter(b, slot)
        return carry

    lax.fori_loop(0, T_STEPS, step, 0)
    drain_scatter((T_STEPS - 1) % 2, (T_STEPS - 1) % 3)
    plsc.subcore_barrier()
    pltpu.sync_copy(agg_sh.at[pl.ds(s * NPS, NPS)],
                    out_hbm.at[c, pl.ds(s * NPS, NPS)])


def _sc_conv(p0, p1, e, ei3, zeros):
    mesh = plsc.VectorSubcoreMesh(core_axis_name="c", subcore_axis_name="s")
    f = pl.kernel(
        _sc_conv_body,
        out_type=jax.ShapeDtypeStruct((NC, NPAD, MSG), jnp.float32),
        mesh=mesh,
        scratch_types=[
            pltpu.VMEM_SHARED((NPAD, MSG), jnp.float32),
            pltpu.VMEM((3, NT, K), jnp.int32),
            pltpu.VMEM((3, NT, K), jnp.int32),
            pltpu.VMEM((2, SUP, MSG), jnp.bfloat16),
            pltpu.VMEM((2, SUP, MSG), jnp.bfloat16),
            pltpu.VMEM((2, SUP // 8, 128), jnp.float32),
            pltpu.VMEM((2, SUP, MSG), jnp.float32),
            pltpu.SemaphoreType.DMA((3,)),
            pltpu.SemaphoreType.DMA((2,)),
            pltpu.SemaphoreType.DMA((2,)),
        ],
        compiler_params=pltpu.CompilerParams(use_tc_tiling_on_sc=False,
                                             needs_layout_passes=False),
    )
    return f(p0, p1, e, ei3, zeros)


_UBLK = 2000


def _node_up_body(x_ref, agg_ref, wna_ref, wnb_ref, bn_ref, wa2_ref, wb2_ref,
                  h_ref, p0_ref, p1_ref):
    aggs = agg_ref[0] + agg_ref[1]
    h = jnp.dot(x_ref[...], wna_ref[...], preferred_element_type=jnp.float32)
    h += jnp.dot(aggs, wnb_ref[...], preferred_element_type=jnp.float32)
    h = jnp.maximum(h + bn_ref[...], 0.0)
    h_ref[...] = h
    p0_ref[...] = jnp.dot(h, wa2_ref[...], preferred_element_type=jnp.float32).astype(jnp.bfloat16)
    p1_ref[...] = jnp.dot(h, wb2_ref[...], preferred_element_type=jnp.float32).astype(jnp.bfloat16)


def _node_update(x, agg, wnat, wnbt, bn, wa2t, wb2t):
    nblk = N_NODES // _UBLK
    return pl.pallas_call(
        _node_up_body,
        grid=(nblk,),
        in_specs=[
            pl.BlockSpec((_UBLK, D_FEAT), lambda i: (i, 0)),
            pl.BlockSpec((NC, _UBLK, MSG), lambda i: (0, i, 0)),
            pl.BlockSpec((D_FEAT, D_FEAT), lambda i: (0, 0)),
            pl.BlockSpec((MSG, D_FEAT), lambda i: (0, 0)),
            pl.BlockSpec((1, D_FEAT), lambda i: (0, 0)),
            pl.BlockSpec((D_FEAT, MSG), lambda i: (0, 0)),
            pl.BlockSpec((D_FEAT, MSG), lambda i: (0, 0)),
        ],
        out_specs=[
            pl.BlockSpec((_UBLK, D_FEAT), lambda i: (i, 0)),
            pl.BlockSpec((_UBLK, MSG), lambda i: (i, 0)),
            pl.BlockSpec((_UBLK, MSG), lambda i: (i, 0)),
        ],
        out_shape=[
            jax.ShapeDtypeStruct((N_NODES, D_FEAT), jnp.float32),
            jax.ShapeDtypeStruct((N_NODES, MSG), jnp.bfloat16),
            jax.ShapeDtypeStruct((N_NODES, MSG), jnp.bfloat16),
        ],
    )(x, agg, wnat, wnbt, bn, wa2t, wb2t)


_PBLK = 2000


def _pool_body(x_ref, agg_ref, batch_ref, wna_ref, wnb_ref, bn_ref,
               wg1_ref, bg1_ref, wg2_ref, bg2_ref, z_ref, sum_acc, cnt_acc):
    i = pl.program_id(0)
    aggs = agg_ref[0] + agg_ref[1]
    h = jnp.dot(x_ref[...], wna_ref[...], preferred_element_type=jnp.float32)
    h += jnp.dot(aggs, wnb_ref[...], preferred_element_type=jnp.float32)
    h = jnp.maximum(h + bn_ref[...], 0.0)

    b = batch_ref[0, 0, :]
    gids = lax.broadcasted_iota(jnp.int32, (N_GRAPHS, _PBLK), 0)
    onehot = (gids == b[None, :]).astype(jnp.float32)

    @pl.when(i == 0)
    def _init():
        sum_acc[...] = jnp.zeros_like(sum_acc)
        cnt_acc[...] = jnp.zeros_like(cnt_acc)

    sum_acc[...] += jnp.dot(onehot, h, preferred_element_type=jnp.float32)
    cnt_acc[...] += jnp.sum(onehot, axis=1, keepdims=True)

    @pl.when(i == pl.num_programs(0) - 1)
    def _final():
        means = sum_acc[...] / jnp.clip(cnt_acc[...], 1.0, None)
        g = jnp.dot(means, wg1_ref[...], preferred_element_type=jnp.float32)
        g = jnp.maximum(g + bg1_ref[...], 0.0)
        z = jnp.dot(g, wg2_ref[...], preferred_element_type=jnp.float32)
        z_ref[...] = z + bg2_ref[...]


def _pool_mlp(x, agg, batch3d, wnat, wnbt, bn, wg1t, bg1, wg2t, bg2):
    nblk = N_NODES // _PBLK
    return pl.pallas_call(
        _pool_body,
        grid=(nblk,),
        in_specs=[
            pl.BlockSpec((_PBLK, D_FEAT), lambda i: (i, 0)),
            pl.BlockSpec((NC, _PBLK, MSG), lambda i: (0, i, 0)),
            pl.BlockSpec((1, 1, _PBLK), lambda i: (i, 0, 0)),
            pl.BlockSpec((D_FEAT, D_FEAT), lambda i: (0, 0)),
            pl.BlockSpec((MSG, D_FEAT), lambda i: (0, 0)),
            pl.BlockSpec((1, D_FEAT), lambda i: (0, 0)),
            pl.BlockSpec((D_FEAT, HID), lambda i: (0, 0)),
            pl.BlockSpec((1, HID), lambda i: (0, 0)),
            pl.BlockSpec((HID, OUT), lambda i: (0, 0)),
            pl.BlockSpec((1, OUT), lambda i: (0, 0)),
        ],
        out_specs=pl.BlockSpec((N_GRAPHS, OUT), lambda i: (0, 0)),
        out_shape=jax.ShapeDtypeStruct((N_GRAPHS, OUT), jnp.float32),
        scratch_shapes=[
            pltpu.VMEM((N_GRAPHS, D_FEAT), jnp.float32),
            pltpu.VMEM((N_GRAPHS, 1), jnp.float32),
        ],
    )(x, agg, batch3d, wnat, wnbt, bn, wg1t, bg1, wg2t, bg2)


def kernel(x, edge_index, edge_attr, batch,
           We1, be1, Wn1, bn1, We2, be2, Wn2, bn2, Wg1, bg1, Wg2, bg2):
    ei3 = edge_index.reshape(2, ROWS2, K)

    wa1t = We1[:, :D_FEAT].T
    wb1t = We1[:, D_FEAT:2 * D_FEAT].T
    wc1t = We1[:, 2 * D_FEAT:].T
    wa2t = We2[:, :D_FEAT].T
    wb2t = We2[:, D_FEAT:2 * D_FEAT].T
    wc2t = We2[:, 2 * D_FEAT:].T
    wna1t = Wn1[:, :D_FEAT].T
    wnb1t = Wn1[:, D_FEAT:].T
    wna2t = Wn2[:, :D_FEAT].T
    wnb2t = Wn2[:, D_FEAT:].T
    wg1t = Wg1.T
    wg2t = Wg2.T

    perm = jnp.concatenate([jnp.arange(0, MSG, 2), jnp.arange(1, MSG, 2)])
    wnb1tp = wnb1t[perm, :]
    wnb2tp = wnb2t[perm, :]
    eye8 = jnp.eye(8, dtype=jnp.float32)
    w1lo = jnp.kron(eye8, wc1t[:, 0::2])
    w1hi = jnp.kron(eye8, wc1t[:, 1::2])
    w2lo = jnp.kron(eye8, wc2t[:, 0::2])
    w2hi = jnp.kron(eye8, wc2t[:, 1::2])
    b1lo = jnp.tile(be1[0::2], 8).reshape(1, 128)
    b1hi = jnp.tile(be1[1::2], 8).reshape(1, 128)
    b2lo = jnp.tile(be2[0::2], 8).reshape(1, 128)
    b2hi = jnp.tile(be2[1::2], 8).reshape(1, 128)
    bn1r = bn1.reshape(1, D_FEAT)
    bn2r = bn2.reshape(1, D_FEAT)
    bg1r = bg1.reshape(1, HID)
    bg2r = bg2.reshape(1, OUT)

    zeros = jnp.zeros((NPAD, MSG), jnp.float32)
    batch3d = batch.reshape(N_NODES // _PBLK, 1, _PBLK)
    ea8 = edge_attr.reshape(N_EDGES // 8, 8 * D_EDGE)

    e1, e2, p0, p1 = _edge_pre(ea8, x, w1lo, b1lo, w1hi, b1hi,
                               w2lo, b2lo, w2hi, b2hi, wa1t, wb1t)
    agg1 = _sc_conv(p0, p1, e1, ei3, zeros)
    h1, q0, q1 = _node_update(x, agg1, wna1t, wnb1tp, bn1r, wa2t, wb2t)

    agg2 = _sc_conv(q0, q1, e2, ei3, zeros)

    z = _pool_mlp(h1, agg2, batch3d, wna2t, wnb2tp, bn2r, wg1t, bg1r, wg2t, bg2r)
    return z


# speedup vs baseline: 11.8437x; 1.0119x over previous
import functools

import jax
import jax.numpy as jnp
from jax import lax
from jax.experimental import pallas as pl
from jax.experimental.pallas import tpu as pltpu
from jax.experimental.pallas import tpu_sc as plsc

N_NODES = 10000
N_EDGES = 320000
D_FEAT = 128
D_EDGE = 16
MSG = 32
HID = 128
OUT = 16
N_GRAPHS = 64

NC = 2
NS = 16
NW = NC * NS
EPW = N_EDGES // NW
K = 400
NPAD = 10240
NPS = NPAD // NS


EROWS = N_EDGES // 8
_EBLK = 2000


def _pack_pair(lo, hi):
    lo16 = lax.bitcast_convert_type(lo.astype(jnp.bfloat16), jnp.uint16).astype(jnp.uint32)
    hi16 = lax.bitcast_convert_type(hi.astype(jnp.bfloat16), jnp.uint16).astype(jnp.uint32)
    return lax.bitcast_convert_type((hi16 << 16) | lo16, jnp.float32)


def _edge_pre_body(ea_ref, x_ref, w1l_ref, b1l_ref, w1h_ref, b1h_ref,
                   w2l_ref, b2l_ref, w2h_ref, b2h_ref, wa_ref, wb_ref,
                   e1_ref, e2_ref, p0_ref, p1_ref):
    ea = ea_ref[...]

    def half(w_ref, b_ref):
        return jnp.dot(ea, w_ref[...], preferred_element_type=jnp.float32) + b_ref[...]

    e1_ref[...] = _pack_pair(half(w1l_ref, b1l_ref), half(w1h_ref, b1h_ref))
    e2_ref[...] = _pack_pair(half(w2l_ref, b2l_ref), half(w2h_ref, b2h_ref))

    @pl.when(pl.program_id(0) < N_NODES // _NBLK)
    def _():
        xv = x_ref[...]
        p0_ref[...] = jnp.dot(xv, wa_ref[...], preferred_element_type=jnp.float32).astype(jnp.bfloat16)
        p1_ref[...] = jnp.dot(xv, wb_ref[...], preferred_element_type=jnp.float32).astype(jnp.bfloat16)


def _edge_pre(ea8, x, *wb):
    nblk = EROWS // _EBLK
    wspec = pl.BlockSpec((128, 128), lambda i: (0, 0))
    bspec = pl.BlockSpec((1, 128), lambda i: (0, 0))
    nlast = N_NODES // _NBLK - 1
    return pl.pallas_call(
        _edge_pre_body,
        grid=(nblk,),
        in_specs=[pl.BlockSpec((_EBLK, 128), lambda i: (i, 0)),
                  pl.BlockSpec((_NBLK, D_FEAT), lambda i: (jnp.minimum(i, nlast), 0))]
                 + [wspec, bspec] * 4
                 + [pl.BlockSpec((D_FEAT, MSG), lambda i: (0, 0))] * 2,
        out_specs=[
            pl.BlockSpec((_EBLK, 128), lambda i: (i, 0)),
            pl.BlockSpec((_EBLK, 128), lambda i: (i, 0)),
            pl.BlockSpec((_NBLK, MSG), lambda i: (jnp.minimum(i, nlast), 0)),
            pl.BlockSpec((_NBLK, MSG), lambda i: (jnp.minimum(i, nlast), 0)),
        ],
        out_shape=[
            jax.ShapeDtypeStruct((EROWS, 128), jnp.float32),
            jax.ShapeDtypeStruct((EROWS, 128), jnp.float32),
            jax.ShapeDtypeStruct((N_NODES, MSG), jnp.bfloat16),
            jax.ShapeDtypeStruct((N_NODES, MSG), jnp.bfloat16),
        ],
    )(ea8, x, *wb)


_NBLK = 2000


SUP = 400
NT = SUP // K
T_STEPS = EPW // SUP
ROWS2 = N_EDGES // K


def _sc_conv_body(p0_hbm, p1_hbm, e_hbm, ei_hbm, z_hbm, out_hbm,
                  agg_sh, idx0_v, idx1_v, g0_v, g1_v, ev_v, m_v,
                  sem_i, sem_g, sem_s):
    c = lax.axis_index("c")
    s = lax.axis_index("s")
    wid = c * NS + s

    pltpu.sync_copy(z_hbm.at[pl.ds(s * NPS, NPS)], agg_sh.at[pl.ds(s * NPS, NPS)])
    plsc.subcore_barrier()

    irow0 = wid * (EPW // K)
    base0 = wid * EPW

    def issue_idx(t, slot):
        r = irow0 + t * NT
        pltpu.async_copy(ei_hbm.at[0, pl.ds(r, NT)], idx0_v.at[slot], sem_i.at[slot])
        pltpu.async_copy(ei_hbm.at[1, pl.ds(r, NT)], idx1_v.at[slot], sem_i.at[slot])

    def drain_idx(t, slot):
        r = irow0 + t * NT
        pltpu.make_async_copy(ei_hbm.at[0, pl.ds(r, NT)], idx0_v.at[slot], sem_i.at[slot]).wait()
        pltpu.make_async_copy(ei_hbm.at[1, pl.ds(r, NT)], idx1_v.at[slot], sem_i.at[slot]).wait()

    def issue_fetch(t, b, slot):
        erow = (base0 + t * SUP) // 8
        pltpu.async_copy(e_hbm.at[pl.ds(erow, SUP // 8)], ev_v.at[b], sem_g.at[b])
        for j in range(NT):
            sl = pl.ds(j * K, K)
            pltpu.async_copy(p0_hbm.at[idx0_v.at[slot, j]], g0_v.at[b, sl], sem_g.at[b])
            pltpu.async_copy(p1_hbm.at[idx1_v.at[slot, j]], g1_v.at[b, sl], sem_g.at[b])

    def drain_fetch(t, b):
        erow = (base0 + t * SUP) // 8
        pltpu.make_async_copy(e_hbm.at[pl.ds(erow, SUP // 8)], ev_v.at[b], sem_g.at[b]).wait()
        pltpu.make_async_copy(p0_hbm.at[pl.ds(0, SUP)], g0_v.at[b], sem_g.at[b]).wait()
        pltpu.make_async_copy(p1_hbm.at[pl.ds(0, SUP)], g1_v.at[b], sem_g.at[b]).wait()

    def issue_scatter(b, slot):
        for j in range(NT):
            sl = pl.ds(j * K, K)
            pltpu.make_async_copy(m_v.at[b, sl], agg_sh.at[idx0_v.at[slot, j]],
                                  sem_s.at[b]).start(add=True)

    def drain_scatter(b, slot):
        for j in range(NT):
            sl = pl.ds(j * K, K)
            pltpu.make_async_copy(m_v.at[b, sl], agg_sh.at[idx0_v.at[slot, j]],
                                  sem_s.at[b]).wait()

    issue_idx(0, 0)
    issue_idx(1, 1)
    drain_idx(0, 0)
    issue_fetch(0, 0, 0)

    def step(t, carry):
        b = t % 2
        slot = t % 3

        @pl.when(t >= 1)
        def _():
            drain_scatter(1 - b, (t - 1) % 3)

        @pl.when(t + 2 < T_STEPS)
        def _():
            issue_idx(t + 2, (t + 2) % 3)

        @pl.when(t + 1 < T_STEPS)
        def _():
            drain_idx(t + 1, (t + 1) % 3)
            issue_fetch(t + 1, 1 - b, (t + 1) % 3)

        drain_fetch(t, b)

        def row4(u, carry2):
            for k in range(4):
                r = u * 4 + k
                er = r >> 3
                ec = (r & 7) * 16
                x0a, x0b = plsc.unpack(g0_v[b, r, :], format=plsc.PackFormat.INTERLEAVED)
                x1a, x1b = plsc.unpack(g1_v[b, r, :], format=plsc.PackFormat.INTERLEAVED)
                ew = plsc.bitcast(ev_v[b, er, pl.ds(ec, 16)], jnp.bfloat16)
                ea_, eb_ = plsc.unpack(ew, format=plsc.PackFormat.INTERLEAVED)
                m_v[b, r, pl.ds(0, 16)] = jnp.maximum(x0a + x1a + ea_, 0.0)
                m_v[b, r, pl.ds(16, 16)] = jnp.maximum(x0b + x1b + eb_, 0.0)
            return carry2

        lax.fori_loop(0, SUP // 4, row4, 0)
        issue_scatter(b, slot)
        return carry

    lax.fori_loop(0, T_STEPS, step, 0)
    drain_scatter((T_STEPS - 1) % 2, (T_STEPS - 1) % 3)
    plsc.subcore_barrier()
    pltpu.sync_copy(agg_sh.at[pl.ds(s * NPS, NPS)],
                    out_hbm.at[c, pl.ds(s * NPS, NPS)])


def _sc_conv(p0, p1, e, ei3, zeros):
    mesh = plsc.VectorSubcoreMesh(core_axis_name="c", subcore_axis_name="s")
    f = pl.kernel(
        _sc_conv_body,
        out_type=jax.ShapeDtypeStruct((NC, NPAD, MSG), jnp.float32),
        mesh=mesh,
        scratch_types=[
            pltpu.VMEM_SHARED((NPAD, MSG), jnp.float32),
            pltpu.VMEM((3, NT, K), jnp.int32),
            pltpu.VMEM((3, NT, K), jnp.int32),
            pltpu.VMEM((2, SUP, MSG), jnp.bfloat16),
            pltpu.VMEM((2, SUP, MSG), jnp.bfloat16),
            pltpu.VMEM((2, SUP // 8, 128), jnp.float32),
            pltpu.VMEM((2, SUP, MSG), jnp.float32),
            pltpu.SemaphoreType.DMA((3,)),
            pltpu.SemaphoreType.DMA((2,)),
            pltpu.SemaphoreType.DMA((2,)),
        ],
        compiler_params=pltpu.CompilerParams(use_tc_tiling_on_sc=False,
                                             needs_layout_passes=False),
    )
    return f(p0, p1, e, ei3, zeros)


_UBLK = 2000


def _node_up_body(x_ref, agg_ref, wna_ref, wnb_ref, bn_ref, wa2_ref, wb2_ref,
                  h_ref, p0_ref, p1_ref):
    aggs = agg_ref[0] + agg_ref[1]
    h = jnp.dot(x_ref[...], wna_ref[...], preferred_element_type=jnp.float32)
    h += jnp.dot(aggs, wnb_ref[...], preferred_element_type=jnp.float32)
    h = jnp.maximum(h + bn_ref[...], 0.0)
    h_ref[...] = h
    p0_ref[...] = jnp.dot(h, wa2_ref[...], preferred_element_type=jnp.float32).astype(jnp.bfloat16)
    p1_ref[...] = jnp.dot(h, wb2_ref[...], preferred_element_type=jnp.float32).astype(jnp.bfloat16)


def _node_update(x, agg, wnat, wnbt, bn, wa2t, wb2t):
    nblk = N_NODES // _UBLK
    return pl.pallas_call(
        _node_up_body,
        grid=(nblk,),
        in_specs=[
            pl.BlockSpec((_UBLK, D_FEAT), lambda i: (i, 0)),
            pl.BlockSpec((NC, _UBLK, MSG), lambda i: (0, i, 0)),
            pl.BlockSpec((D_FEAT, D_FEAT), lambda i: (0, 0)),
            pl.BlockSpec((MSG, D_FEAT), lambda i: (0, 0)),
            pl.BlockSpec((1, D_FEAT), lambda i: (0, 0)),
            pl.BlockSpec((D_FEAT, MSG), lambda i: (0, 0)),
            pl.BlockSpec((D_FEAT, MSG), lambda i: (0, 0)),
        ],
        out_specs=[
            pl.BlockSpec((_UBLK, D_FEAT), lambda i: (i, 0)),
            pl.BlockSpec((_UBLK, MSG), lambda i: (i, 0)),
            pl.BlockSpec((_UBLK, MSG), lambda i: (i, 0)),
        ],
        out_shape=[
            jax.ShapeDtypeStruct((N_NODES, D_FEAT), jnp.float32),
            jax.ShapeDtypeStruct((N_NODES, MSG), jnp.bfloat16),
            jax.ShapeDtypeStruct((N_NODES, MSG), jnp.bfloat16),
        ],
    )(x, agg, wnat, wnbt, bn, wa2t, wb2t)


_PBLK = 2000


def _pool_body(x_ref, agg_ref, batch_ref, wna_ref, wnb_ref, bn_ref,
               wg1_ref, bg1_ref, wg2_ref, bg2_ref, z_ref, sum_acc, cnt_acc):
    i = pl.program_id(0)
    aggs = agg_ref[0] + agg_ref[1]
    h = jnp.dot(x_ref[...], wna_ref[...], preferred_element_type=jnp.float32)
    h += jnp.dot(aggs, wnb_ref[...], preferred_element_type=jnp.float32)
    h = jnp.maximum(h + bn_ref[...], 0.0)

    b = batch_ref[0, 0, :]
    gids = lax.broadcasted_iota(jnp.int32, (N_GRAPHS, _PBLK), 0)
    onehot = (gids == b[None, :]).astype(jnp.float32)

    @pl.when(i == 0)
    def _init():
        sum_acc[...] = jnp.zeros_like(sum_acc)
        cnt_acc[...] = jnp.zeros_like(cnt_acc)

    sum_acc[...] += jnp.dot(onehot, h, preferred_element_type=jnp.float32)
    cnt_acc[...] += jnp.sum(onehot, axis=1, keepdims=True)

    @pl.when(i == pl.num_programs(0) - 1)
    def _final():
        means = sum_acc[...] / jnp.clip(cnt_acc[...], 1.0, None)
        g = jnp.dot(means, wg1_ref[...], preferred_element_type=jnp.float32)
        g = jnp.maximum(g + bg1_ref[...], 0.0)
        z = jnp.dot(g, wg2_ref[...], preferred_element_type=jnp.float32)
        z_ref[...] = z + bg2_ref[...]


def _pool_mlp(x, agg, batch3d, wnat, wnbt, bn, wg1t, bg1, wg2t, bg2):
    nblk = N_NODES // _PBLK
    return pl.pallas_call(
        _pool_body,
        grid=(nblk,),
        in_specs=[
            pl.BlockSpec((_PBLK, D_FEAT), lambda i: (i, 0)),
            pl.BlockSpec((NC, _PBLK, MSG), lambda i: (0, i, 0)),
            pl.BlockSpec((1, 1, _PBLK), lambda i: (i, 0, 0)),
            pl.BlockSpec((D_FEAT, D_FEAT), lambda i: (0, 0)),
            pl.BlockSpec((MSG, D_FEAT), lambda i: (0, 0)),
            pl.BlockSpec((1, D_FEAT), lambda i: (0, 0)),
            pl.BlockSpec((D_FEAT, HID), lambda i: (0, 0)),
            pl.BlockSpec((1, HID), lambda i: (0, 0)),
            pl.BlockSpec((HID, OUT), lambda i: (0, 0)),
            pl.BlockSpec((1, OUT), lambda i: (0, 0)),
        ],
        out_specs=pl.BlockSpec((N_GRAPHS, OUT), lambda i: (0, 0)),
        out_shape=jax.ShapeDtypeStruct((N_GRAPHS, OUT), jnp.float32),
        scratch_shapes=[
            pltpu.VMEM((N_GRAPHS, D_FEAT), jnp.float32),
            pltpu.VMEM((N_GRAPHS, 1), jnp.float32),
        ],
    )(x, agg, batch3d, wnat, wnbt, bn, wg1t, bg1, wg2t, bg2)


def kernel(x, edge_index, edge_attr, batch,
           We1, be1, Wn1, bn1, We2, be2, Wn2, bn2, Wg1, bg1, Wg2, bg2):
    ei3 = edge_index.reshape(2, ROWS2, K)

    wa1t = We1[:, :D_FEAT].T
    wb1t = We1[:, D_FEAT:2 * D_FEAT].T
    wc1t = We1[:, 2 * D_FEAT:].T
    wa2t = We2[:, :D_FEAT].T
    wb2t = We2[:, D_FEAT:2 * D_FEAT].T
    wc2t = We2[:, 2 * D_FEAT:].T
    wna1t = Wn1[:, :D_FEAT].T
    wnb1t = Wn1[:, D_FEAT:].T
    wna2t = Wn2[:, :D_FEAT].T
    wnb2t = Wn2[:, D_FEAT:].T
    wg1t = Wg1.T
    wg2t = Wg2.T

    perm = jnp.concatenate([jnp.arange(0, MSG, 2), jnp.arange(1, MSG, 2)])
    wnb1tp = wnb1t[perm, :]
    wnb2tp = wnb2t[perm, :]
    eye8 = jnp.eye(8, dtype=jnp.float32)
    w1lo = jnp.kron(eye8, wc1t[:, 0::2])
    w1hi = jnp.kron(eye8, wc1t[:, 1::2])
    w2lo = jnp.kron(eye8, wc2t[:, 0::2])
    w2hi = jnp.kron(eye8, wc2t[:, 1::2])
    b1lo = jnp.tile(be1[0::2], 8).reshape(1, 128)
    b1hi = jnp.tile(be1[1::2], 8).reshape(1, 128)
    b2lo = jnp.tile(be2[0::2], 8).reshape(1, 128)
    b2hi = jnp.tile(be2[1::2], 8).reshape(1, 128)
    bn1r = bn1.reshape(1, D_FEAT)
    bn2r = bn2.reshape(1, D_FEAT)
    bg1r = bg1.reshape(1, HID)
    bg2r = bg2.reshape(1, OUT)

    zeros = jnp.zeros((NPAD, MSG), jnp.float32)
    batch3d = batch.reshape(N_NODES // _PBLK, 1, _PBLK)
    ea8 = edge_attr.reshape(N_EDGES // 8, 8 * D_EDGE)

    e1, e2, p0, p1 = _edge_pre(ea8, x, w1lo, b1lo, w1hi, b1hi,
                               w2lo, b2lo, w2hi, b2hi, wa1t, wb1t)
    agg1 = _sc_conv(p0, p1, e1, ei3, zeros)
    h1, q0, q1 = _node_update(x, agg1, wna1t, wnb1tp, bn1r, wa2t, wb2t)

    agg2 = _sc_conv(q0, q1, e2, ei3, zeros)

    z = _pool_mlp(h1, agg2, batch3d, wna2t, wnb2tp, bn2r, wg1t, bg1r, wg2t, bg2r)
    return z
